# Initial kernel scaffold; baseline (speedup 1.0000x reference)
import jax
import jax.numpy as jnp
from jax.experimental import pallas as pl


def kernel(Z, Rij, idx_i, idx_j, idx_i_triples, idx_j_triples, idx_k_triples, radial_mu, elem_w_radial, angular_mu, elem_w_angular, symfunc_mean, symfunc_stddev):
    raise NotImplementedError("write your pallas kernel here")


# speedup vs baseline: 3.7390x; 3.7390x over previous
import functools
import math

import jax
import jax.numpy as jnp
from jax import lax
from jax.experimental import pallas as pl
from jax.experimental.pallas import tpu as pltpu
from jax.experimental.pallas import tpu_sc as plsc

N = 100000
E = 1600000
T = 2000000
NB_R = 32
NB_A = 16
N_ELEM = 100
RC = 5.0
ETA_R = 4.0
ETA_A = 2.0

NC = 2
NS = 16
NW = NC * NS
NBW = 8

_COS_COEF = [(-1.0) ** k * math.pi ** (2 * k) / math.factorial(2 * k)
             for k in range(8)]

_EC = 256
_NEC = E // _EC
_RSTEPS = (_NEC + NW - 1) // NW
_TC_ = 128
_NTC = T // _TC_
_ASTEPS = (_NTC + NW - 1) // NW
_FR = 200
_NFC = N // _FR
_NFS = (_NFC + NS - 1) // NS


def _rsqrt_newton(s):
    i = lax.bitcast_convert_type(s, jnp.int32)
    i = jnp.int32(0x5F3759DF) - (i >> 1)
    y = lax.bitcast_convert_type(i, jnp.float32)
    for _ in range(3):
        y = y * (1.5 - 0.5 * s * y * y)
    return y


def _fc_poly(d):
    v = d * d * (1.0 / (RC * RC))
    p = jnp.full((16,), _COS_COEF[7], jnp.float32)
    for c in _COS_COEF[6::-1]:
        p = p * v + c
    return jnp.where(d < RC, 0.5 * (p + 1.0), jnp.zeros((16,), jnp.float32))


def _fill_zero(zbuf):
    lanes = jnp.arange(16, dtype=jnp.int32)
    rowpat = lanes // NBW
    colpat = lanes % NBW
    z16 = jnp.zeros((16,), jnp.float32)
    for i in range(_FR * NBW // 16):
        plsc.store_scatter(zbuf, [rowpat + i * (16 // NBW), colpat], z16)


def _zero_acc(acc, zbuf, s):
    for k in range(_NFS):
        f = s + k * NS

        @pl.when(f < _NFC)
        def _go():
            o = pl.multiple_of(f * _FR, _FR)
            pltpu.sync_copy(zbuf, acc.at[pl.ds(o, _FR), :])


def _flush_acc(acc, fbuf, s, out_ref):
    for k in range(_NFS):
        f = s + k * NS

        @pl.when(f < _NFC)
        def _go():
            o = pl.multiple_of(f * _FR, _FR)
            pltpu.sync_copy(acc.at[pl.ds(o, _FR), :], fbuf)
            pltpu.sync_copy(fbuf, out_ref.at[pl.ds(o, _FR), :])


def _radial_body(rxh, ryh, rzh, ii2, ij2, z, wr, mur, out, rec,
                 acc, wr_v, mu_v, rx, ry, rz, ij_v, ii_v, zjv,
                 stage, rec_st, fbuf, zbuf, semz):
    c = lax.axis_index("c")
    s = lax.axis_index("s")
    wid = s * NC + c

    pltpu.sync_copy(wr, wr_v)
    pltpu.sync_copy(mur, mu_v)
    _fill_zero(zbuf)
    lanes = jnp.arange(16, dtype=jnp.int32)

    def chunk(g, h, mu16):
        base = g * _EC
        pltpu.sync_copy(rxh.at[pl.ds(base, _EC)], rx)
        pltpu.sync_copy(ryh.at[pl.ds(base, _EC)], ry)
        pltpu.sync_copy(rzh.at[pl.ds(base, _EC)], rz)
        pltpu.sync_copy(ij2.at[pl.ds(g * 2, 2)], ij_v)
        pltpu.sync_copy(ii2.at[pl.ds(g * 2, 2)], ii_v)
        gz0 = pltpu.async_copy(z.at[ij_v.at[0]], zjv.at[0], semz)
        gz1 = pltpu.async_copy(z.at[ij_v.at[1]], zjv.at[1], semz)
        gz0.wait()
        gz1.wait()
        for sub in range(2):
            for grp in range(8):
                o = sub * 128 + grp * 16
                rows = lanes + o
                x = rx[pl.ds(o, 16)]
                y = ry[pl.ds(o, 16)]
                q = rz[pl.ds(o, 16)]
                s2 = x * x + y * y + q * q
                d = s2 * _rsqrt_newton(s2)
                fc = _fc_poly(d)
                zj = zjv[sub, pl.ds(grp * 16, 16)]

                @pl.when(h == 0)
                def _wrec():
                    for f, val in enumerate(
                            (x, y, q, d, fc,
                             lax.bitcast_convert_type(zj, jnp.float32))):
                        plsc.store_scatter(
                            rec_st, [rows, jnp.full((16,), f, jnp.int32)], val)
                for b in range(NBW):
                    w = plsc.load_gather(wr_v, [zj * NB_R + (h * NBW + b)])
                    t = d - mu16[b]
                    val = jnp.exp(-ETA_R * t * t) * fc * w
                    plsc.store_scatter(
                        stage, [rows, jnp.full((16,), b, jnp.int32)], val)
            pltpu.sync_copy(stage.at[pl.ds(sub * 128, 128), :],
                            acc.at[ii_v.at[sub]], add=True)

        @pl.when(h == 0)
        def _drec():
            pltpu.sync_copy(rec_st, rec.at[pl.ds(base, _EC), :])

    def half_loop(h, carry):
        _zero_acc(acc, zbuf, s)
        plsc.subcore_barrier()
        mu16 = mu_v[pl.ds(pl.multiple_of(h * NBW, NBW), 16)]

        def step(t, cc):
            g = wid + NW * t

            @pl.when(g < _NEC)
            def _go():
                chunk(g, h, mu16)
            return cc
        lax.fori_loop(0, _RSTEPS, step, 0)
        plsc.subcore_barrier()
        _flush_acc(acc, fbuf, s, out.at[h, c])
        plsc.subcore_barrier()
        return carry

    lax.fori_loop(0, NB_R // NBW, half_loop, 0)


def _angular_body(rec, ij2, ik2, ii2, wa, mua, outa,
                  acc, wa_v, mu_v, ij_v, ik_v, ii_v, recj, reck,
                  stage, fbuf, zbuf, semj, semk):
    c = lax.axis_index("c")
    s = lax.axis_index("s")
    wid = s * NC + c

    pltpu.sync_copy(wa, wa_v)
    pltpu.sync_copy(mua, mu_v)
    _fill_zero(zbuf)
    lanes = jnp.arange(16, dtype=jnp.int32)

    def chunk(g, h, mu16):
        pltpu.sync_copy(ij2.at[pl.ds(g, 1)], ij_v)
        pltpu.sync_copy(ik2.at[pl.ds(g, 1)], ik_v)
        pltpu.sync_copy(ii2.at[pl.ds(g, 1)], ii_v)
        cj = pltpu.async_copy(rec.at[ij_v.at[0]], recj, semj)
        ck = pltpu.async_copy(rec.at[ik_v.at[0]], reck, semk)
        cj.wait()
        ck.wait()
        for grp in range(8):
            o = grp * 16
            rows = lanes + o

            def fld(r, f):
                return plsc.load_gather(
                    r, [rows, jnp.full((16,), f, jnp.int32)])
            xj, yj, qj, dj, fcj, zjf = (fld(recj, f) for f in range(6))
            xk, yk, qk, dk, fck, zkf = (fld(reck, f) for f in range(6))
            dot = xj * xk + yj * yk + qj * qk
            cos_t = dot / (dj * dk)
            a = 1.0 + cos_t
            a2 = a * a
            a4 = a2 * a2
            ang = (a4 * a4) * (2.0 ** (1.0 - 8.0))
            davg = 0.5 * (dj + dk)
            scale = ang * fcj * fck
            zj = lax.bitcast_convert_type(zjf, jnp.int32)
            zk = lax.bitcast_convert_type(zkf, jnp.int32)
            for b in range(NBW):
                wj = plsc.load_gather(wa_v, [zj * NB_A + (h * NBW + b)])
                wk = plsc.load_gather(wa_v, [zk * NB_A + (h * NBW + b)])
                t = davg - mu16[b]
                val = scale * jnp.exp(-ETA_A * t * t) * wj * wk
                plsc.store_scatter(
                    stage, [rows, jnp.full((16,), b, jnp.int32)], val)
        pltpu.sync_copy(stage, acc.at[ii_v.at[0]], add=True)

    def half_loop(h, carry):
        _zero_acc(acc, zbuf, s)
        plsc.subcore_barrier()
        mu16 = mu_v[pl.ds(pl.multiple_of(h * NBW, NBW), 16)]

        def step(t, cc):
            g = wid + NW * t

            @pl.when(g < _NTC)
            def _go():
                chunk(g, h, mu16)
            return cc
        lax.fori_loop(0, _ASTEPS, step, 0)
        plsc.subcore_barrier()
        _flush_acc(acc, fbuf, s, outa.at[h, c])
        plsc.subcore_barrier()
        return carry

    lax.fori_loop(0, NB_A // NBW, half_loop, 0)


_mesh = plsc.VectorSubcoreMesh(core_axis_name="c", subcore_axis_name="s")

_radial = functools.partial(
    pl.kernel,
    out_type=[
        jax.ShapeDtypeStruct((NB_R // NBW, NC, N, NBW), jnp.float32),
        jax.ShapeDtypeStruct((E, 8), jnp.float32),
    ],
    mesh=_mesh,
    compiler_params=pltpu.CompilerParams(needs_layout_passes=False, use_tc_tiling_on_sc=False),
    scratch_types=[
        pltpu.VMEM_SHARED((N, NBW), jnp.float32),
        pltpu.VMEM((N_ELEM * NB_R,), jnp.float32),
        pltpu.VMEM((NB_R + 8,), jnp.float32),
        pltpu.VMEM((_EC,), jnp.float32),
        pltpu.VMEM((_EC,), jnp.float32),
        pltpu.VMEM((_EC,), jnp.float32),
        pltpu.VMEM((2, 128), jnp.int32),
        pltpu.VMEM((2, 128), jnp.int32),
        pltpu.VMEM((2, 128), jnp.int32),
        pltpu.VMEM((_EC, NBW), jnp.float32),
        pltpu.VMEM((_EC, 8), jnp.float32),
        pltpu.VMEM((_FR, NBW), jnp.float32),
        pltpu.VMEM((_FR, NBW), jnp.float32),
        pltpu.SemaphoreType.DMA,
    ],
)(_radial_body)

_angular = functools.partial(
    pl.kernel,
    out_type=[
        jax.ShapeDtypeStruct((NB_A // NBW, NC, N, NBW), jnp.float32),
    ],
    mesh=_mesh,
    compiler_params=pltpu.CompilerParams(needs_layout_passes=False, use_tc_tiling_on_sc=False),
    scratch_types=[
        pltpu.VMEM_SHARED((N, NBW), jnp.float32),
        pltpu.VMEM((N_ELEM * NB_A,), jnp.float32),
        pltpu.VMEM((NB_A + 8,), jnp.float32),
        pltpu.VMEM((1, 128), jnp.int32),
        pltpu.VMEM((1, 128), jnp.int32),
        pltpu.VMEM((1, 128), jnp.int32),
        pltpu.VMEM((_TC_, 8), jnp.float32),
        pltpu.VMEM((_TC_, 8), jnp.float32),
        pltpu.VMEM((_TC_, NBW), jnp.float32),
        pltpu.VMEM((_FR, NBW), jnp.float32),
        pltpu.VMEM((_FR, NBW), jnp.float32),
        pltpu.SemaphoreType.DMA,
        pltpu.SemaphoreType.DMA,
    ],
)(_angular_body)


def _combine_body(rad, ang, mean, std, out):
    parts = [rad[p, 0] + rad[p, 1] for p in range(NB_R // NBW)]
    parts += [ang[p, 0] + ang[p, 1] for p in range(NB_A // NBW)]
    x = jnp.concatenate(parts, axis=1)
    out[...] = (x - mean[...]) / std[...]


_BN = 1000
_combine = pl.pallas_call(
    _combine_body,
    grid=(N // _BN,),
    in_specs=[
        pl.BlockSpec((NB_R // NBW, NC, _BN, NBW), lambda i: (0, 0, i, 0)),
        pl.BlockSpec((NB_A // NBW, NC, _BN, NBW), lambda i: (0, 0, i, 0)),
        pl.BlockSpec((1, NB_R + NB_A), lambda i: (0, 0)),
        pl.BlockSpec((1, NB_R + NB_A), lambda i: (0, 0)),
    ],
    out_specs=pl.BlockSpec((_BN, NB_R + NB_A), lambda i: (i, 0)),
    out_shape=jax.ShapeDtypeStruct((N, NB_R + NB_A), jnp.float32),
)


@jax.jit
def kernel(Z, Rij, idx_i, idx_j, idx_i_triples, idx_j_triples, idx_k_triples,
           radial_mu, elem_w_radial, angular_mu, elem_w_angular,
           symfunc_mean, symfunc_stddev):
    rijt = Rij.astype(jnp.float32)
    rxh, ryh, rzh = rijt[:, 0], rijt[:, 1], rijt[:, 2]
    z = Z.astype(jnp.int32)
    ii2 = idx_i.astype(jnp.int32).reshape(E // 128, 128)
    ij2 = idx_j.astype(jnp.int32).reshape(E // 128, 128)
    iit2 = idx_i_triples.astype(jnp.int32).reshape(T // 128, 128)
    ijt2 = idx_j_triples.astype(jnp.int32).reshape(T // 128, 128)
    ikt2 = idx_k_triples.astype(jnp.int32).reshape(T // 128, 128)
    mur = jnp.pad(radial_mu.astype(jnp.float32), (0, 8))
    mua = jnp.pad(angular_mu.astype(jnp.float32), (0, 8))

    rad, rec = _radial(rxh, ryh, rzh, ii2, ij2, z,
                       elem_w_radial.reshape(-1).astype(jnp.float32), mur)
    (ang,) = _angular(rec, ijt2, ikt2, iit2,
                      elem_w_angular.reshape(-1).astype(jnp.float32), mua)
    return _combine(rad, ang,
                    symfunc_mean.astype(jnp.float32),
                    symfunc_stddev.astype(jnp.float32))


# speedup vs baseline: 5.8835x; 1.5735x over previous
import functools
import math

import jax
import jax.numpy as jnp
from jax import lax
from jax.experimental import pallas as pl
from jax.experimental.pallas import tpu as pltpu
from jax.experimental.pallas import tpu_sc as plsc

N = 100000
E = 1600000
T = 2000000
NB_R = 32
NB_A = 16
N_ELEM = 100
RC = 5.0
ETA_R = 4.0
ETA_A = 2.0

NC = 2
NS = 16
NW = NC * NS
NBW = 8

_COS_COEF = [(-1.0) ** k * math.pi ** (2 * k) / math.factorial(2 * k)
             for k in range(8)]

_EC = 256
_NEC = E // _EC
_RSTEPS = 98
_TC_ = 128
_NTC = T // _TC_
_ASTEPS = 245
_FR = 200
_NFC = N // _FR
_NFS = (_NFC + NS - 1) // NS


def _rsqrt_newton(s):
    i = lax.bitcast_convert_type(s, jnp.int32)
    i = jnp.int32(0x5F3759DF) - (i >> 1)
    y = lax.bitcast_convert_type(i, jnp.float32)
    for _ in range(3):
        y = y * (1.5 - 0.5 * s * y * y)
    return y


def _fc_poly(d):
    v = d * d * (1.0 / (RC * RC))
    p = jnp.full((16,), _COS_COEF[7], jnp.float32)
    for c in _COS_COEF[6::-1]:
        p = p * v + c
    return jnp.where(d < RC, 0.5 * (p + 1.0), jnp.zeros((16,), jnp.float32))


def _fill_zero(zbuf):
    lanes = jnp.arange(16, dtype=jnp.int32)
    rowpat = lanes // NBW
    colpat = lanes % NBW
    z16 = jnp.zeros((16,), jnp.float32)
    for i in range(_FR * NBW // 16):
        plsc.store_scatter(zbuf, [rowpat + i * (16 // NBW), colpat], z16)


def _zero_acc(acc, zbuf, s):
    for k in range(_NFS):
        f = s + k * NS

        @pl.when(f < _NFC)
        def _go():
            o = pl.multiple_of(f * _FR, _FR)
            pltpu.sync_copy(zbuf, acc.at[pl.ds(o, _FR), :])


def _flush_acc(acc, fbuf, s, out_ref):
    for k in range(_NFS):
        f = s + k * NS

        @pl.when(f < _NFC)
        def _go():
            o = pl.multiple_of(f * _FR, _FR)
            pltpu.sync_copy(acc.at[pl.ds(o, _FR), :], fbuf)
            pltpu.sync_copy(fbuf, out_ref.at[pl.ds(o, _FR), :])


def _radial_body(rxyz, eidx, z, wr, mur, out, rec,
                 acc, wr_v, mu_v,
                 in0, in1, id0, id1, zj0, zj1,
                 stage, rec_st, fbuf, zbuf,
                 semi0, semi1, semz0, semz1):
    c = lax.axis_index("c")
    s = lax.axis_index("s")
    wid = s * NC + c
    inb = (in0, in1)
    idb = (id0, id1)
    zjb = (zj0, zj1)
    semi = (semi0, semi1)
    semz = (semz0, semz1)

    pltpu.sync_copy(wr, wr_v)
    pltpu.sync_copy(mur, mu_v)
    _fill_zero(zbuf)
    lanes = jnp.arange(16, dtype=jnp.int32)

    def issue(g, b):
        gc = jnp.minimum(g, _NEC - 1)
        pltpu.sync_copy(eidx.at[gc], idb[b])
        pltpu.async_copy(rxyz.at[gc], inb[b], semi[b])
        pltpu.async_copy(z.at[idb[b].at[0, 0]], zjb[b].at[0], semz[b])
        pltpu.async_copy(z.at[idb[b].at[1, 0]], zjb[b].at[1], semz[b])

    def wait(b):
        pltpu.make_async_copy(rxyz.at[0], inb[b], semi[b]).wait()
        pltpu.make_async_copy(z.at[idb[b].at[0, 0]], zjb[b].at[0],
                              semz[b]).wait()
        pltpu.make_async_copy(z.at[idb[b].at[1, 0]], zjb[b].at[1],
                              semz[b]).wait()

    def compute(g, h, mu16, b):
        ib, db, zb = inb[b], idb[b], zjb[b]
        for sub in range(2):
            for grp in range(8):
                o = sub * 128 + grp * 16
                rows = lanes + o
                x = ib[0, pl.ds(o, 16)]
                y = ib[1, pl.ds(o, 16)]
                q = ib[2, pl.ds(o, 16)]
                s2 = x * x + y * y + q * q
                d = s2 * _rsqrt_newton(s2)
                fc = _fc_poly(d)
                zj = zb[sub, pl.ds(grp * 16, 16)]

                @pl.when(h == 0)
                def _wrec():
                    for f, val in enumerate(
                            (x, y, q, d, fc,
                             lax.bitcast_convert_type(zj, jnp.float32))):
                        plsc.store_scatter(
                            rec_st, [rows, jnp.full((16,), f, jnp.int32)], val)
                for bb in range(NBW):
                    w = plsc.load_gather(wr_v, [zj * NB_R + (h * NBW + bb)])
                    t = d - mu16[bb]
                    val = jnp.exp(-ETA_R * t * t) * fc * w
                    plsc.store_scatter(
                        stage, [rows, jnp.full((16,), bb, jnp.int32)], val)
            pltpu.sync_copy(stage.at[pl.ds(sub * 128, 128), :],
                            acc.at[db.at[sub, 1]], add=True)

        @pl.when(h == 0)
        def _drec():
            pltpu.sync_copy(rec_st, rec.at[pl.ds(g * _EC, _EC), :])

    def half_loop(h, carry):
        _zero_acc(acc, zbuf, s)
        plsc.subcore_barrier()
        mu16 = mu_v[pl.ds(pl.multiple_of(h * NBW, NBW), 16)]
        issue(wid, 0)

        def step(t2, cc):
            for b in range(2):
                t = 2 * t2 + b
                g = wid + NW * t
                issue(wid + NW * (t + 1), 1 - b)
                wait(b)

                @pl.when(g < _NEC)
                def _go():
                    compute(g, h, mu16, b)
            return cc
        lax.fori_loop(0, _RSTEPS, step, 0)
        wait(0)
        plsc.subcore_barrier()
        _flush_acc(acc, fbuf, s, out.at[h, c])
        plsc.subcore_barrier()
        return carry

    lax.fori_loop(0, NB_R // NBW, half_loop, 0)


def _angular_body(rec, aidx, wa, mua, outa,
                  acc, wa_v, mu_v,
                  id0, id1, rj0, rj1, rk0, rk1,
                  stage, fbuf, zbuf,
                  sem0, sem1):
    c = lax.axis_index("c")
    s = lax.axis_index("s")
    wid = s * NC + c
    idb = (id0, id1)
    rjb = (rj0, rj1)
    rkb = (rk0, rk1)
    sems = (sem0, sem1)

    pltpu.sync_copy(wa, wa_v)
    pltpu.sync_copy(mua, mu_v)
    _fill_zero(zbuf)
    lanes = jnp.arange(16, dtype=jnp.int32)

    def issue(g, b):
        gc = jnp.minimum(g, _NTC - 1)
        pltpu.sync_copy(aidx.at[gc], idb[b])
        pltpu.async_copy(rec.at[idb[b].at[0]], rjb[b], sems[b])
        pltpu.async_copy(rec.at[idb[b].at[1]], rkb[b], sems[b])

    def wait(b):
        pltpu.make_async_copy(rec.at[idb[b].at[0]], rjb[b], sems[b]).wait()
        pltpu.make_async_copy(rec.at[idb[b].at[1]], rkb[b], sems[b]).wait()

    def compute(h, mu16, b):
        recj, reck = rjb[b], rkb[b]
        for grp in range(8):
            o = grp * 16
            rows = lanes + o

            def fld(r, f):
                return plsc.load_gather(
                    r, [rows, jnp.full((16,), f, jnp.int32)])
            xj, yj, qj, dj, fcj, zjf = (fld(recj, f) for f in range(6))
            xk, yk, qk, dk, fck, zkf = (fld(reck, f) for f in range(6))
            dot = xj * xk + yj * yk + qj * qk
            cos_t = dot / (dj * dk)
            a = 1.0 + cos_t
            a2 = a * a
            a4 = a2 * a2
            ang = (a4 * a4) * (2.0 ** (1.0 - 8.0))
            davg = 0.5 * (dj + dk)
            scale = ang * fcj * fck
            zj = lax.bitcast_convert_type(zjf, jnp.int32)
            zk = lax.bitcast_convert_type(zkf, jnp.int32)
            for bb in range(NBW):
                wj = plsc.load_gather(wa_v, [zj * NB_A + (h * NBW + bb)])
                wk = plsc.load_gather(wa_v, [zk * NB_A + (h * NBW + bb)])
                t = davg - mu16[bb]
                val = scale * jnp.exp(-ETA_A * t * t) * wj * wk
                plsc.store_scatter(
                    stage, [rows, jnp.full((16,), bb, jnp.int32)], val)
        pltpu.sync_copy(stage, acc.at[idb[b].at[2]], add=True)

    def half_loop(h, carry):
        _zero_acc(acc, zbuf, s)
        plsc.subcore_barrier()
        mu16 = mu_v[pl.ds(pl.multiple_of(h * NBW, NBW), 16)]
        issue(wid, 0)

        def step(t2, cc):
            for b in range(2):
                t = 2 * t2 + b
                g = wid + NW * t
                issue(wid + NW * (t + 1), 1 - b)
                wait(b)

                @pl.when(g < _NTC)
                def _go():
                    compute(h, mu16, b)
            return cc
        lax.fori_loop(0, _ASTEPS, step, 0)
        wait(0)
        plsc.subcore_barrier()
        _flush_acc(acc, fbuf, s, outa.at[h, c])
        plsc.subcore_barrier()
        return carry

    lax.fori_loop(0, NB_A // NBW, half_loop, 0)


_mesh = plsc.VectorSubcoreMesh(core_axis_name="c", subcore_axis_name="s")

_radial = functools.partial(
    pl.kernel,
    out_type=[
        jax.ShapeDtypeStruct((NB_R // NBW, NC, N, NBW), jnp.float32),
        jax.ShapeDtypeStruct((E, 8), jnp.float32),
    ],
    mesh=_mesh,
    compiler_params=pltpu.CompilerParams(
        needs_layout_passes=False, use_tc_tiling_on_sc=False),
    scratch_types=[
        pltpu.VMEM_SHARED((N, NBW), jnp.float32),
        pltpu.VMEM((N_ELEM * NB_R,), jnp.float32),
        pltpu.VMEM((NB_R + 8,), jnp.float32),
        pltpu.VMEM((3, _EC), jnp.float32),
        pltpu.VMEM((3, _EC), jnp.float32),
        pltpu.VMEM((2, 2, 128), jnp.int32),
        pltpu.VMEM((2, 2, 128), jnp.int32),
        pltpu.VMEM((2, 128), jnp.int32),
        pltpu.VMEM((2, 128), jnp.int32),
        pltpu.VMEM((_EC, NBW), jnp.float32),
        pltpu.VMEM((_EC, 8), jnp.float32),
        pltpu.VMEM((_FR, NBW), jnp.float32),
        pltpu.VMEM((_FR, NBW), jnp.float32),
        pltpu.SemaphoreType.DMA,
        pltpu.SemaphoreType.DMA,
        pltpu.SemaphoreType.DMA,
        pltpu.SemaphoreType.DMA,
    ],
)(_radial_body)

_angular = functools.partial(
    pl.kernel,
    out_type=[
        jax.ShapeDtypeStruct((NB_A // NBW, NC, N, NBW), jnp.float32),
    ],
    mesh=_mesh,
    compiler_params=pltpu.CompilerParams(
        needs_layout_passes=False, use_tc_tiling_on_sc=False),
    scratch_types=[
        pltpu.VMEM_SHARED((N, NBW), jnp.float32),
        pltpu.VMEM((N_ELEM * NB_A,), jnp.float32),
        pltpu.VMEM((NB_A + 8,), jnp.float32),
        pltpu.VMEM((3, 128), jnp.int32),
        pltpu.VMEM((3, 128), jnp.int32),
        pltpu.VMEM((_TC_, 8), jnp.float32),
        pltpu.VMEM((_TC_, 8), jnp.float32),
        pltpu.VMEM((_TC_, 8), jnp.float32),
        pltpu.VMEM((_TC_, 8), jnp.float32),
        pltpu.VMEM((_TC_, NBW), jnp.float32),
        pltpu.VMEM((_FR, NBW), jnp.float32),
        pltpu.VMEM((_FR, NBW), jnp.float32),
        pltpu.SemaphoreType.DMA,
        pltpu.SemaphoreType.DMA,
    ],
)(_angular_body)


def _combine_body(rad, ang, mean, std, out):
    parts = [rad[p, 0] + rad[p, 1] for p in range(NB_R // NBW)]
    parts += [ang[p, 0] + ang[p, 1] for p in range(NB_A // NBW)]
    x = jnp.concatenate(parts, axis=1)
    out[...] = (x - mean[...]) / std[...]


_BN = 1000
_combine = pl.pallas_call(
    _combine_body,
    grid=(N // _BN,),
    in_specs=[
        pl.BlockSpec((NB_R // NBW, NC, _BN, NBW), lambda i: (0, 0, i, 0)),
        pl.BlockSpec((NB_A // NBW, NC, _BN, NBW), lambda i: (0, 0, i, 0)),
        pl.BlockSpec((1, NB_R + NB_A), lambda i: (0, 0)),
        pl.BlockSpec((1, NB_R + NB_A), lambda i: (0, 0)),
    ],
    out_specs=pl.BlockSpec((_BN, NB_R + NB_A), lambda i: (i, 0)),
    out_shape=jax.ShapeDtypeStruct((N, NB_R + NB_A), jnp.float32),
)


@jax.jit
def kernel(Z, Rij, idx_i, idx_j, idx_i_triples, idx_j_triples, idx_k_triples,
           radial_mu, elem_w_radial, angular_mu, elem_w_angular,
           symfunc_mean, symfunc_stddev):
    rijt = Rij.astype(jnp.float32)
    rxyz = rijt.reshape(E // _EC, _EC, 3).transpose(0, 2, 1)
    z = Z.astype(jnp.int32)
    eidx = jnp.stack(
        [idx_j.astype(jnp.int32).reshape(E // _EC, 2, 128),
         idx_i.astype(jnp.int32).reshape(E // _EC, 2, 128)], axis=2)
    aidx = jnp.stack(
        [idx_j_triples.astype(jnp.int32).reshape(T // _TC_, 128),
         idx_k_triples.astype(jnp.int32).reshape(T // _TC_, 128),
         idx_i_triples.astype(jnp.int32).reshape(T // _TC_, 128)], axis=1)
    mur = jnp.pad(radial_mu.astype(jnp.float32), (0, 8))
    mua = jnp.pad(angular_mu.astype(jnp.float32), (0, 8))

    rad, rec = _radial(rxyz, eidx, z,
                       elem_w_radial.reshape(-1).astype(jnp.float32), mur)
    (ang,) = _angular(rec, aidx,
                      elem_w_angular.reshape(-1).astype(jnp.float32), mua)
    return _combine(rad, ang,
                    symfunc_mean.astype(jnp.float32),
                    symfunc_stddev.astype(jnp.float32))


# speedup vs baseline: 6.6416x; 1.1288x over previous
import functools
import math

import jax
import jax.numpy as jnp
from jax import lax
from jax.experimental import pallas as pl
from jax.experimental.pallas import tpu as pltpu
from jax.experimental.pallas import tpu_sc as plsc

N = 100000
E = 1600000
T = 2000000
NB_R = 32
NB_A = 16
N_ELEM = 100
RC = 5.0
ETA_R = 4.0
ETA_A = 2.0

NC = 2
NS = 16
NW = NC * NS
NBW = 8

_COS_COEF = [(-1.0) ** k * math.pi ** (2 * k) / math.factorial(2 * k)
             for k in range(8)]

_EC = 256
_NEC = E // _EC
_RSTEPS = 98
_TC_ = 128
_NTC = T // _TC_
_ASTEPS = 245
_FR = 200
_NFC = N // _FR
_NFS = (_NFC + NS - 1) // NS


def _rsqrt_newton(s):
    i = lax.bitcast_convert_type(s, jnp.int32)
    i = jnp.int32(0x5F3759DF) - (i >> 1)
    y = lax.bitcast_convert_type(i, jnp.float32)
    for _ in range(3):
        y = y * (1.5 - 0.5 * s * y * y)
    return y


def _fc_poly(d):
    v = d * d * (1.0 / (RC * RC))
    p = jnp.full((16,), _COS_COEF[7], jnp.float32)
    for c in _COS_COEF[6::-1]:
        p = p * v + c
    return jnp.where(d < RC, 0.5 * (p + 1.0), jnp.zeros((16,), jnp.float32))


def _fill_zero(zbuf):
    lanes = jnp.arange(16, dtype=jnp.int32)
    rowpat = lanes // NBW
    colpat = lanes % NBW
    z16 = jnp.zeros((16,), jnp.float32)
    for i in range(_FR * NBW // 16):
        plsc.store_scatter(zbuf, [rowpat + i * (16 // NBW), colpat], z16)


def _zero_acc(acc, zbuf, s):
    for k in range(_NFS):
        f = s + k * NS

        @pl.when(f < _NFC)
        def _go():
            o = pl.multiple_of(f * _FR, _FR)
            pltpu.sync_copy(zbuf, acc.at[pl.ds(o, _FR), :])


def _flush_acc(acc, fbuf, s, out_ref):
    for k in range(_NFS):
        f = s + k * NS

        @pl.when(f < _NFC)
        def _go():
            o = pl.multiple_of(f * _FR, _FR)
            pltpu.sync_copy(acc.at[pl.ds(o, _FR), :], fbuf)
            pltpu.sync_copy(fbuf, out_ref.at[pl.ds(o, _FR), :])


def _radial_body(rxyz, eidx, z, wr, mur, out, rec,
                 acc, wr_v, mu_v,
                 in0, in1, id0, id1, zj0, zj1, ri0, ri1,
                 stage, rec_st, fbuf, zbuf,
                 semi0, semi1, semz0, semz1, semr0, semr1):
    c = lax.axis_index("c")
    s = lax.axis_index("s")
    wid = s * NC + c
    inb = (in0, in1)
    idb = (id0, id1)
    zjb = (zj0, zj1)
    rib = (ri0, ri1)
    semi = (semi0, semi1)
    semz = (semz0, semz1)
    semr = (semr0, semr1)

    pltpu.sync_copy(wr, wr_v)
    pltpu.sync_copy(mur, mu_v)
    _fill_zero(zbuf)
    lanes = jnp.arange(16, dtype=jnp.int32)

    def issue(g, b, h):
        gc = jnp.minimum(g, _NEC - 1)
        pltpu.sync_copy(eidx.at[gc], idb[b])

        @pl.when(h == 0)
        def _i0():
            pltpu.async_copy(rxyz.at[gc], inb[b], semi[b])
            pltpu.async_copy(z.at[idb[b].at[0, 0]], zjb[b].at[0], semz[b])
            pltpu.async_copy(z.at[idb[b].at[1, 0]], zjb[b].at[1], semz[b])

        @pl.when(h != 0)
        def _i1():
            pltpu.async_copy(rec.at[pl.ds(gc * _EC, _EC), :], rib[b], semr[b])

    def wait(b, h):
        @pl.when(h == 0)
        def _w0():
            pltpu.make_async_copy(rxyz.at[0], inb[b], semi[b]).wait()
            pltpu.make_async_copy(z.at[idb[b].at[0, 0]], zjb[b].at[0],
                                  semz[b]).wait()
            pltpu.make_async_copy(z.at[idb[b].at[1, 0]], zjb[b].at[1],
                                  semz[b]).wait()

        @pl.when(h != 0)
        def _w1():
            pltpu.make_async_copy(rec.at[pl.ds(0, _EC), :], rib[b],
                                  semr[b]).wait()

    def basis(d, fc, zj, rows, h, mu16):
        for bb in range(NBW):
            w = plsc.load_gather(wr_v, [zj * NB_R + (h * NBW + bb)])
            t = d - mu16[bb]
            val = jnp.exp(-ETA_R * t * t) * fc * w
            plsc.store_scatter(
                stage, [rows, jnp.full((16,), bb, jnp.int32)], val)

    def compute(g, h, mu16, b):
        ib, db, zb, rb = inb[b], idb[b], zjb[b], rib[b]
        for sub in range(2):
            @pl.when(h == 0)
            def _c0():
                for grp in range(8):
                    o = sub * 128 + grp * 16
                    rows = lanes + o
                    x = ib[0, pl.ds(o, 16)]
                    y = ib[1, pl.ds(o, 16)]
                    q = ib[2, pl.ds(o, 16)]
                    s2 = x * x + y * y + q * q
                    d = s2 * _rsqrt_newton(s2)
                    fc = _fc_poly(d)
                    zj = zb[sub, pl.ds(grp * 16, 16)]
                    for f, val in enumerate(
                            (x, y, q, d, fc,
                             lax.bitcast_convert_type(zj, jnp.float32))):
                        plsc.store_scatter(
                            rec_st, [rows, jnp.full((16,), f, jnp.int32)], val)
                    basis(d, fc, zj, rows, h, mu16)

            @pl.when(h != 0)
            def _c1():
                for grp in range(8):
                    o = sub * 128 + grp * 16
                    rows = lanes + o
                    d = plsc.load_gather(
                        rb, [rows, jnp.full((16,), 3, jnp.int32)])
                    fc = plsc.load_gather(
                        rb, [rows, jnp.full((16,), 4, jnp.int32)])
                    zjf = plsc.load_gather(
                        rb, [rows, jnp.full((16,), 5, jnp.int32)])
                    zj = lax.bitcast_convert_type(zjf, jnp.int32)
                    basis(d, fc, zj, rows, h, mu16)
            pltpu.sync_copy(stage.at[pl.ds(sub * 128, 128), :],
                            acc.at[db.at[sub, 1]], add=True)

        @pl.when(h == 0)
        def _drec():
            pltpu.sync_copy(rec_st, rec.at[pl.ds(g * _EC, _EC), :])

    def half_loop(h, carry):
        _zero_acc(acc, zbuf, s)
        plsc.subcore_barrier()
        mu16 = mu_v[pl.ds(pl.multiple_of(h * NBW, NBW), 16)]
        issue(wid, 0, h)

        def step(t2, cc):
            for b in range(2):
                t = 2 * t2 + b
                g = wid + NW * t
                issue(wid + NW * (t + 1), 1 - b, h)
                wait(b, h)

                @pl.when(g < _NEC)
                def _go():
                    compute(g, h, mu16, b)
            return cc
        lax.fori_loop(0, _RSTEPS, step, 0)
        wait(0, h)
        plsc.subcore_barrier()
        _flush_acc(acc, fbuf, s, out.at[h, c])
        plsc.subcore_barrier()
        return carry

    lax.fori_loop(0, NB_R // NBW, half_loop, 0)


def _angular_body(rec, aidx, wa, mua, outa,
                  acc, wa_v, mu_v,
                  id0, id1, rj0, rj1, rk0, rk1,
                  stage, fbuf, zbuf,
                  sem0, sem1):
    c = lax.axis_index("c")
    s = lax.axis_index("s")
    wid = s * NC + c
    idb = (id0, id1)
    rjb = (rj0, rj1)
    rkb = (rk0, rk1)
    sems = (sem0, sem1)

    pltpu.sync_copy(wa, wa_v)
    pltpu.sync_copy(mua, mu_v)
    _fill_zero(zbuf)
    lanes = jnp.arange(16, dtype=jnp.int32)

    def issue(g, b):
        gc = jnp.minimum(g, _NTC - 1)
        pltpu.sync_copy(aidx.at[gc], idb[b])
        pltpu.async_copy(rec.at[idb[b].at[0]], rjb[b], sems[b])
        pltpu.async_copy(rec.at[idb[b].at[1]], rkb[b], sems[b])

    def wait(b):
        pltpu.make_async_copy(rec.at[idb[b].at[0]], rjb[b], sems[b]).wait()
        pltpu.make_async_copy(rec.at[idb[b].at[1]], rkb[b], sems[b]).wait()

    def compute(h, mu16, b):
        recj, reck = rjb[b], rkb[b]
        for grp in range(8):
            o = grp * 16
            rows = lanes + o

            def fld(r, f):
                return plsc.load_gather(
                    r, [rows, jnp.full((16,), f, jnp.int32)])
            xj, yj, qj, dj, fcj, zjf = (fld(recj, f) for f in range(6))
            xk, yk, qk, dk, fck, zkf = (fld(reck, f) for f in range(6))
            dot = xj * xk + yj * yk + qj * qk
            cos_t = dot / (dj * dk)
            a = 1.0 + cos_t
            a2 = a * a
            a4 = a2 * a2
            ang = (a4 * a4) * (2.0 ** (1.0 - 8.0))
            davg = 0.5 * (dj + dk)
            scale = ang * fcj * fck
            zj = lax.bitcast_convert_type(zjf, jnp.int32)
            zk = lax.bitcast_convert_type(zkf, jnp.int32)
            for bb in range(NBW):
                wj = plsc.load_gather(wa_v, [zj * NB_A + (h * NBW + bb)])
                wk = plsc.load_gather(wa_v, [zk * NB_A + (h * NBW + bb)])
                t = davg - mu16[bb]
                val = scale * jnp.exp(-ETA_A * t * t) * wj * wk
                plsc.store_scatter(
                    stage, [rows, jnp.full((16,), bb, jnp.int32)], val)
        pltpu.sync_copy(stage, acc.at[idb[b].at[2]], add=True)

    def half_loop(h, carry):
        _zero_acc(acc, zbuf, s)
        plsc.subcore_barrier()
        mu16 = mu_v[pl.ds(pl.multiple_of(h * NBW, NBW), 16)]
        issue(wid, 0)

        def step(t2, cc):
            for b in range(2):
                t = 2 * t2 + b
                g = wid + NW * t
                issue(wid + NW * (t + 1), 1 - b)
                wait(b)

                @pl.when(g < _NTC)
                def _go():
                    compute(h, mu16, b)
            return cc
        lax.fori_loop(0, _ASTEPS, step, 0)
        wait(0)
        plsc.subcore_barrier()
        _flush_acc(acc, fbuf, s, outa.at[h, c])
        plsc.subcore_barrier()
        return carry

    lax.fori_loop(0, NB_A // NBW, half_loop, 0)


_mesh = plsc.VectorSubcoreMesh(core_axis_name="c", subcore_axis_name="s")

_radial = functools.partial(
    pl.kernel,
    out_type=[
        jax.ShapeDtypeStruct((NB_R // NBW, NC, N, NBW), jnp.float32),
        jax.ShapeDtypeStruct((E, 8), jnp.float32),
    ],
    mesh=_mesh,
    compiler_params=pltpu.CompilerParams(
        needs_layout_passes=False, use_tc_tiling_on_sc=False),
    scratch_types=[
        pltpu.VMEM_SHARED((N, NBW), jnp.float32),
        pltpu.VMEM((N_ELEM * NB_R,), jnp.float32),
        pltpu.VMEM((NB_R + 8,), jnp.float32),
        pltpu.VMEM((3, _EC), jnp.float32),
        pltpu.VMEM((3, _EC), jnp.float32),
        pltpu.VMEM((2, 2, 128), jnp.int32),
        pltpu.VMEM((2, 2, 128), jnp.int32),
        pltpu.VMEM((2, 128), jnp.int32),
        pltpu.VMEM((2, 128), jnp.int32),
        pltpu.VMEM((_EC, 8), jnp.float32),
        pltpu.VMEM((_EC, 8), jnp.float32),
        pltpu.VMEM((_EC, NBW), jnp.float32),
        pltpu.VMEM((_EC, 8), jnp.float32),
        pltpu.VMEM((_FR, NBW), jnp.float32),
        pltpu.VMEM((_FR, NBW), jnp.float32),
        pltpu.SemaphoreType.DMA,
        pltpu.SemaphoreType.DMA,
        pltpu.SemaphoreType.DMA,
        pltpu.SemaphoreType.DMA,
        pltpu.SemaphoreType.DMA,
        pltpu.SemaphoreType.DMA,
    ],
)(_radial_body)

_angular = functools.partial(
    pl.kernel,
    out_type=[
        jax.ShapeDtypeStruct((NB_A // NBW, NC, N, NBW), jnp.float32),
    ],
    mesh=_mesh,
    compiler_params=pltpu.CompilerParams(
        needs_layout_passes=False, use_tc_tiling_on_sc=False),
    scratch_types=[
        pltpu.VMEM_SHARED((N, NBW), jnp.float32),
        pltpu.VMEM((N_ELEM * NB_A,), jnp.float32),
        pltpu.VMEM((NB_A + 8,), jnp.float32),
        pltpu.VMEM((3, 128), jnp.int32),
        pltpu.VMEM((3, 128), jnp.int32),
        pltpu.VMEM((_TC_, 8), jnp.float32),
        pltpu.VMEM((_TC_, 8), jnp.float32),
        pltpu.VMEM((_TC_, 8), jnp.float32),
        pltpu.VMEM((_TC_, 8), jnp.float32),
        pltpu.VMEM((_TC_, NBW), jnp.float32),
        pltpu.VMEM((_FR, NBW), jnp.float32),
        pltpu.VMEM((_FR, NBW), jnp.float32),
        pltpu.SemaphoreType.DMA,
        pltpu.SemaphoreType.DMA,
    ],
)(_angular_body)


def _combine_body(rad, ang, mean, std, out):
    parts = [rad[p, 0] + rad[p, 1] for p in range(NB_R // NBW)]
    parts += [ang[p, 0] + ang[p, 1] for p in range(NB_A // NBW)]
    x = jnp.concatenate(parts, axis=1)
    out[...] = (x - mean[...]) / std[...]


_BN = 1000
_combine = pl.pallas_call(
    _combine_body,
    grid=(N // _BN,),
    in_specs=[
        pl.BlockSpec((NB_R // NBW, NC, _BN, NBW), lambda i: (0, 0, i, 0)),
        pl.BlockSpec((NB_A // NBW, NC, _BN, NBW), lambda i: (0, 0, i, 0)),
        pl.BlockSpec((1, NB_R + NB_A), lambda i: (0, 0)),
        pl.BlockSpec((1, NB_R + NB_A), lambda i: (0, 0)),
    ],
    out_specs=pl.BlockSpec((_BN, NB_R + NB_A), lambda i: (i, 0)),
    out_shape=jax.ShapeDtypeStruct((N, NB_R + NB_A), jnp.float32),
)


@jax.jit
def kernel(Z, Rij, idx_i, idx_j, idx_i_triples, idx_j_triples, idx_k_triples,
           radial_mu, elem_w_radial, angular_mu, elem_w_angular,
           symfunc_mean, symfunc_stddev):
    rijt = Rij.astype(jnp.float32)
    rxyz = rijt.reshape(E // _EC, _EC, 3).transpose(0, 2, 1)
    z = Z.astype(jnp.int32)
    eidx = jnp.stack(
        [idx_j.astype(jnp.int32).reshape(E // _EC, 2, 128),
         idx_i.astype(jnp.int32).reshape(E // _EC, 2, 128)], axis=2)
    aidx = jnp.stack(
        [idx_j_triples.astype(jnp.int32).reshape(T // _TC_, 128),
         idx_k_triples.astype(jnp.int32).reshape(T // _TC_, 128),
         idx_i_triples.astype(jnp.int32).reshape(T // _TC_, 128)], axis=1)
    mur = jnp.pad(radial_mu.astype(jnp.float32), (0, 8))
    mua = jnp.pad(angular_mu.astype(jnp.float32), (0, 8))

    rad, rec = _radial(rxyz, eidx, z,
                       elem_w_radial.reshape(-1).astype(jnp.float32), mur)
    (ang,) = _angular(rec, aidx,
                      elem_w_angular.reshape(-1).astype(jnp.float32), mua)
    return _combine(rad, ang,
                    symfunc_mean.astype(jnp.float32),
                    symfunc_stddev.astype(jnp.float32))


# speedup vs baseline: 6.6432x; 1.0002x over previous
import functools
import math

import jax
import jax.numpy as jnp
from jax import lax
from jax.experimental import pallas as pl
from jax.experimental.pallas import tpu as pltpu
from jax.experimental.pallas import tpu_sc as plsc

N = 100000
E = 1600000
T = 2000000
NB_R = 32
NB_A = 16
N_ELEM = 100
RC = 5.0
ETA_R = 4.0
ETA_A = 2.0

NC = 2
NS = 16
NW = NC * NS
NBW = 8

_COS_COEF = [(-1.0) ** k * math.pi ** (2 * k) / math.factorial(2 * k)
             for k in range(8)]

_EC = 256
_NEC = E // _EC
_RSTEPS = 98
_TC_ = 128
_NTC = T // _TC_
_ASTEPS = 245
_FR = 200
_NFC = N // _FR
_NFS = (_NFC + NS - 1) // NS


def _rsqrt_newton(s):
    i = lax.bitcast_convert_type(s, jnp.int32)
    i = jnp.int32(0x5F3759DF) - (i >> 1)
    y = lax.bitcast_convert_type(i, jnp.float32)
    for _ in range(3):
        y = y * (1.5 - 0.5 * s * y * y)
    return y


def _fc_poly(d):
    v = d * d * (1.0 / (RC * RC))
    p = jnp.full((16,), _COS_COEF[7], jnp.float32)
    for c in _COS_COEF[6::-1]:
        p = p * v + c
    return jnp.where(d < RC, 0.5 * (p + 1.0), jnp.zeros((16,), jnp.float32))


def _fill_zero(zbuf):
    lanes = jnp.arange(16, dtype=jnp.int32)
    rowpat = lanes // NBW
    colpat = lanes % NBW
    z16 = jnp.zeros((16,), jnp.float32)
    for i in range(_FR * NBW // 16):
        plsc.store_scatter(zbuf, [rowpat + i * (16 // NBW), colpat], z16)


def _zero_acc(acc, zbuf, s):
    for k in range(_NFS):
        f = s + k * NS

        @pl.when(f < _NFC)
        def _go():
            o = pl.multiple_of(f * _FR, _FR)
            pltpu.sync_copy(zbuf, acc.at[pl.ds(o, _FR), :])


def _flush_acc(acc, fbuf, s, out_ref):
    for k in range(_NFS):
        f = s + k * NS

        @pl.when(f < _NFC)
        def _go():
            o = pl.multiple_of(f * _FR, _FR)
            pltpu.sync_copy(acc.at[pl.ds(o, _FR), :], fbuf)
            pltpu.sync_copy(fbuf, out_ref.at[pl.ds(o, _FR), :])


def _radial_body(rxyz, eidx, z, wr, mur, out, rec,
                 acc, wr_v, mu_v,
                 in0, in1, id0, id1, zj0, zj1, ri0, ri1,
                 stage, rec_st, fbuf, zbuf,
                 semi0, semi1, semz0, semz1, semr0, semr1, semx0, semx1):
    c = lax.axis_index("c")
    s = lax.axis_index("s")
    wid = s * NC + c
    inb = (in0, in1)
    idb = (id0, id1)
    zjb = (zj0, zj1)
    rib = (ri0, ri1)
    semi = (semi0, semi1)
    semz = (semz0, semz1)
    semr = (semr0, semr1)
    semx = (semx0, semx1)

    pltpu.sync_copy(wr, wr_v)
    pltpu.sync_copy(mur, mu_v)
    _fill_zero(zbuf)
    lanes = jnp.arange(16, dtype=jnp.int32)

    def issue_idx(g, b):
        gc = jnp.minimum(g, _NEC - 1)
        pltpu.async_copy(eidx.at[gc], idb[b], semx[b])

    def wait_idx(b):
        pltpu.make_async_copy(eidx.at[0], idb[b], semx[b]).wait()

    def issue_data(g, b, h):
        gc = jnp.minimum(g, _NEC - 1)

        @pl.when(h == 0)
        def _i0():
            pltpu.async_copy(rxyz.at[gc], inb[b], semi[b])
            pltpu.async_copy(z.at[idb[b].at[0, 0]], zjb[b].at[0], semz[b])
            pltpu.async_copy(z.at[idb[b].at[1, 0]], zjb[b].at[1], semz[b])

        @pl.when(h != 0)
        def _i1():
            pltpu.async_copy(rec.at[pl.ds(gc * _EC, _EC), :], rib[b], semr[b])

    def wait_data(b, h):
        @pl.when(h == 0)
        def _w0():
            pltpu.make_async_copy(rxyz.at[0], inb[b], semi[b]).wait()
            pltpu.make_async_copy(z.at[idb[b].at[0, 0]], zjb[b].at[0],
                                  semz[b]).wait()
            pltpu.make_async_copy(z.at[idb[b].at[1, 0]], zjb[b].at[1],
                                  semz[b]).wait()

        @pl.when(h != 0)
        def _w1():
            pltpu.make_async_copy(rec.at[pl.ds(0, _EC), :], rib[b],
                                  semr[b]).wait()

    def basis(d, fc, zj, rows, h, mu16):
        for bb in range(NBW):
            w = plsc.load_gather(wr_v, [zj * NB_R + (h * NBW + bb)])
            t = d - mu16[bb]
            val = jnp.exp(-ETA_R * t * t) * fc * w
            plsc.store_scatter(
                stage, [rows, jnp.full((16,), bb, jnp.int32)], val)

    def compute(g, h, mu16, b):
        ib, db, zb, rb = inb[b], idb[b], zjb[b], rib[b]
        for sub in range(2):
            @pl.when(h == 0)
            def _c0():
                for grp in range(8):
                    o = sub * 128 + grp * 16
                    rows = lanes + o
                    x = ib[0, pl.ds(o, 16)]
                    y = ib[1, pl.ds(o, 16)]
                    q = ib[2, pl.ds(o, 16)]
                    s2 = x * x + y * y + q * q
                    d = s2 * _rsqrt_newton(s2)
                    fc = _fc_poly(d)
                    zj = zb[sub, pl.ds(grp * 16, 16)]
                    for f, val in enumerate(
                            (x, y, q, d, fc,
                             lax.bitcast_convert_type(zj, jnp.float32))):
                        plsc.store_scatter(
                            rec_st, [rows, jnp.full((16,), f, jnp.int32)], val)
                    basis(d, fc, zj, rows, h, mu16)

            @pl.when(h != 0)
            def _c1():
                for grp in range(8):
                    o = sub * 128 + grp * 16
                    rows = lanes + o
                    d = plsc.load_gather(
                        rb, [rows, jnp.full((16,), 3, jnp.int32)])
                    fc = plsc.load_gather(
                        rb, [rows, jnp.full((16,), 4, jnp.int32)])
                    zjf = plsc.load_gather(
                        rb, [rows, jnp.full((16,), 5, jnp.int32)])
                    zj = lax.bitcast_convert_type(zjf, jnp.int32)
                    basis(d, fc, zj, rows, h, mu16)
            pltpu.sync_copy(stage.at[pl.ds(sub * 128, 128), :],
                            acc.at[db.at[sub, 1]], add=True)

        @pl.when(h == 0)
        def _drec():
            pltpu.sync_copy(rec_st, rec.at[pl.ds(g * _EC, _EC), :])

    def half_loop(h, carry):
        _zero_acc(acc, zbuf, s)
        plsc.subcore_barrier()
        mu16 = mu_v[pl.ds(pl.multiple_of(h * NBW, NBW), 16)]
        pltpu.sync_copy(eidx.at[wid], idb[0])
        issue_data(wid, 0, h)
        issue_idx(wid + NW, 1)

        def step(t2, cc):
            for b in range(2):
                t = 2 * t2 + b
                g = wid + NW * t
                wait_idx(1 - b)
                issue_data(wid + NW * (t + 1), 1 - b, h)
                wait_data(b, h)

                @pl.when(g < _NEC)
                def _go():
                    compute(g, h, mu16, b)
                issue_idx(wid + NW * (t + 2), b)
            return cc
        lax.fori_loop(0, _RSTEPS, step, 0)
        wait_data(0, h)
        wait_idx(1)
        plsc.subcore_barrier()
        _flush_acc(acc, fbuf, s, out.at[h, c])
        plsc.subcore_barrier()
        return carry

    lax.fori_loop(0, NB_R // NBW, half_loop, 0)


def _angular_body(rec, aidx, wa, mua, outa,
                  acc, wa_v, mu_v,
                  id0, id1, rj0, rj1, rk0, rk1,
                  stage, fbuf, zbuf,
                  sem0, sem1, semx0, semx1):
    c = lax.axis_index("c")
    s = lax.axis_index("s")
    wid = s * NC + c
    idb = (id0, id1)
    rjb = (rj0, rj1)
    rkb = (rk0, rk1)
    sems = (sem0, sem1)
    semx = (semx0, semx1)

    pltpu.sync_copy(wa, wa_v)
    pltpu.sync_copy(mua, mu_v)
    _fill_zero(zbuf)
    lanes = jnp.arange(16, dtype=jnp.int32)

    def issue_idx(g, b):
        gc = jnp.minimum(g, _NTC - 1)
        pltpu.async_copy(aidx.at[gc], idb[b], semx[b])

    def wait_idx(b):
        pltpu.make_async_copy(aidx.at[0], idb[b], semx[b]).wait()

    def issue_data(g, b):
        pltpu.async_copy(rec.at[idb[b].at[0]], rjb[b], sems[b])
        pltpu.async_copy(rec.at[idb[b].at[1]], rkb[b], sems[b])

    def wait_data(b):
        pltpu.make_async_copy(rec.at[idb[b].at[0]], rjb[b], sems[b]).wait()
        pltpu.make_async_copy(rec.at[idb[b].at[1]], rkb[b], sems[b]).wait()

    def compute(h, mu16, b):
        recj, reck = rjb[b], rkb[b]
        for grp in range(8):
            o = grp * 16
            rows = lanes + o

            def fld(r, f):
                return plsc.load_gather(
                    r, [rows, jnp.full((16,), f, jnp.int32)])
            xj, yj, qj, dj, fcj, zjf = (fld(recj, f) for f in range(6))
            xk, yk, qk, dk, fck, zkf = (fld(reck, f) for f in range(6))
            dot = xj * xk + yj * yk + qj * qk
            cos_t = dot / (dj * dk)
            a = 1.0 + cos_t
            a2 = a * a
            a4 = a2 * a2
            ang = (a4 * a4) * (2.0 ** (1.0 - 8.0))
            davg = 0.5 * (dj + dk)
            scale = ang * fcj * fck
            zj = lax.bitcast_convert_type(zjf, jnp.int32)
            zk = lax.bitcast_convert_type(zkf, jnp.int32)
            for bb in range(NBW):
                wj = plsc.load_gather(wa_v, [zj * NB_A + (h * NBW + bb)])
                wk = plsc.load_gather(wa_v, [zk * NB_A + (h * NBW + bb)])
                t = davg - mu16[bb]
                val = scale * jnp.exp(-ETA_A * t * t) * wj * wk
                plsc.store_scatter(
                    stage, [rows, jnp.full((16,), bb, jnp.int32)], val)
        pltpu.sync_copy(stage, acc.at[idb[b].at[2]], add=True)

    def half_loop(h, carry):
        _zero_acc(acc, zbuf, s)
        plsc.subcore_barrier()
        mu16 = mu_v[pl.ds(pl.multiple_of(h * NBW, NBW), 16)]
        pltpu.sync_copy(aidx.at[wid], idb[0])
        issue_data(wid, 0)
        issue_idx(wid + NW, 1)

        def step(t2, cc):
            for b in range(2):
                t = 2 * t2 + b
                g = wid + NW * t
                wait_idx(1 - b)
                issue_data(wid + NW * (t + 1), 1 - b)
                wait_data(b)

                @pl.when(g < _NTC)
                def _go():
                    compute(h, mu16, b)
                issue_idx(wid + NW * (t + 2), b)
            return cc
        lax.fori_loop(0, _ASTEPS, step, 0)
        wait_data(0)
        wait_idx(1)
        plsc.subcore_barrier()
        _flush_acc(acc, fbuf, s, outa.at[h, c])
        plsc.subcore_barrier()
        return carry

    lax.fori_loop(0, NB_A // NBW, half_loop, 0)


_mesh = plsc.VectorSubcoreMesh(core_axis_name="c", subcore_axis_name="s")

_radial = functools.partial(
    pl.kernel,
    out_type=[
        jax.ShapeDtypeStruct((NB_R // NBW, NC, N, NBW), jnp.float32),
        jax.ShapeDtypeStruct((E, 8), jnp.float32),
    ],
    mesh=_mesh,
    compiler_params=pltpu.CompilerParams(
        needs_layout_passes=False, use_tc_tiling_on_sc=False),
    scratch_types=[
        pltpu.VMEM_SHARED((N, NBW), jnp.float32),
        pltpu.VMEM((N_ELEM * NB_R,), jnp.float32),
        pltpu.VMEM((NB_R + 8,), jnp.float32),
        pltpu.VMEM((3, _EC), jnp.float32),
        pltpu.VMEM((3, _EC), jnp.float32),
        pltpu.VMEM((2, 2, 128), jnp.int32),
        pltpu.VMEM((2, 2, 128), jnp.int32),
        pltpu.VMEM((2, 128), jnp.int32),
        pltpu.VMEM((2, 128), jnp.int32),
        pltpu.VMEM((_EC, 8), jnp.float32),
        pltpu.VMEM((_EC, 8), jnp.float32),
        pltpu.VMEM((_EC, NBW), jnp.float32),
        pltpu.VMEM((_EC, 8), jnp.float32),
        pltpu.VMEM((_FR, NBW), jnp.float32),
        pltpu.VMEM((_FR, NBW), jnp.float32),
        pltpu.SemaphoreType.DMA,
        pltpu.SemaphoreType.DMA,
        pltpu.SemaphoreType.DMA,
        pltpu.SemaphoreType.DMA,
        pltpu.SemaphoreType.DMA,
        pltpu.SemaphoreType.DMA,
        pltpu.SemaphoreType.DMA,
        pltpu.SemaphoreType.DMA,
    ],
)(_radial_body)

_angular = functools.partial(
    pl.kernel,
    out_type=[
        jax.ShapeDtypeStruct((NB_A // NBW, NC, N, NBW), jnp.float32),
    ],
    mesh=_mesh,
    compiler_params=pltpu.CompilerParams(
        needs_layout_passes=False, use_tc_tiling_on_sc=False),
    scratch_types=[
        pltpu.VMEM_SHARED((N, NBW), jnp.float32),
        pltpu.VMEM((N_ELEM * NB_A,), jnp.float32),
        pltpu.VMEM((NB_A + 8,), jnp.float32),
        pltpu.VMEM((3, 128), jnp.int32),
        pltpu.VMEM((3, 128), jnp.int32),
        pltpu.VMEM((_TC_, 8), jnp.float32),
        pltpu.VMEM((_TC_, 8), jnp.float32),
        pltpu.VMEM((_TC_, 8), jnp.float32),
        pltpu.VMEM((_TC_, 8), jnp.float32),
        pltpu.VMEM((_TC_, NBW), jnp.float32),
        pltpu.VMEM((_FR, NBW), jnp.float32),
        pltpu.VMEM((_FR, NBW), jnp.float32),
        pltpu.SemaphoreType.DMA,
        pltpu.SemaphoreType.DMA,
        pltpu.SemaphoreType.DMA,
        pltpu.SemaphoreType.DMA,
    ],
)(_angular_body)


def _combine_body(rad, ang, mean, std, out):
    parts = [rad[p, 0] + rad[p, 1] for p in range(NB_R // NBW)]
    parts += [ang[p, 0] + ang[p, 1] for p in range(NB_A // NBW)]
    x = jnp.concatenate(parts, axis=1)
    out[...] = (x - mean[...]) / std[...]


_BN = 1000
_combine = pl.pallas_call(
    _combine_body,
    grid=(N // _BN,),
    in_specs=[
        pl.BlockSpec((NB_R // NBW, NC, _BN, NBW), lambda i: (0, 0, i, 0)),
        pl.BlockSpec((NB_A // NBW, NC, _BN, NBW), lambda i: (0, 0, i, 0)),
        pl.BlockSpec((1, NB_R + NB_A), lambda i: (0, 0)),
        pl.BlockSpec((1, NB_R + NB_A), lambda i: (0, 0)),
    ],
    out_specs=pl.BlockSpec((_BN, NB_R + NB_A), lambda i: (i, 0)),
    out_shape=jax.ShapeDtypeStruct((N, NB_R + NB_A), jnp.float32),
)


@jax.jit
def kernel(Z, Rij, idx_i, idx_j, idx_i_triples, idx_j_triples, idx_k_triples,
           radial_mu, elem_w_radial, angular_mu, elem_w_angular,
           symfunc_mean, symfunc_stddev):
    rijt = Rij.astype(jnp.float32)
    rxyz = rijt.reshape(E // _EC, _EC, 3).transpose(0, 2, 1)
    z = Z.astype(jnp.int32)
    eidx = jnp.stack(
        [idx_j.astype(jnp.int32).reshape(E // _EC, 2, 128),
         idx_i.astype(jnp.int32).reshape(E // _EC, 2, 128)], axis=2)
    aidx = jnp.stack(
        [idx_j_triples.astype(jnp.int32).reshape(T // _TC_, 128),
         idx_k_triples.astype(jnp.int32).reshape(T // _TC_, 128),
         idx_i_triples.astype(jnp.int32).reshape(T // _TC_, 128)], axis=1)
    mur = jnp.pad(radial_mu.astype(jnp.float32), (0, 8))
    mua = jnp.pad(angular_mu.astype(jnp.float32), (0, 8))

    rad, rec = _radial(rxyz, eidx, z,
                       elem_w_radial.reshape(-1).astype(jnp.float32), mur)
    (ang,) = _angular(rec, aidx,
                      elem_w_angular.reshape(-1).astype(jnp.float32), mua)
    return _combine(rad, ang,
                    symfunc_mean.astype(jnp.float32),
                    symfunc_stddev.astype(jnp.float32))


# speedup vs baseline: 6.8145x; 1.0258x over previous
import functools
import math

import jax
import jax.numpy as jnp
from jax import lax
from jax.experimental import pallas as pl
from jax.experimental.pallas import tpu as pltpu
from jax.experimental.pallas import tpu_sc as plsc

N = 100000
E = 1600000
T = 2000000
NB_R = 32
NB_A = 16
N_ELEM = 100
RC = 5.0
ETA_R = 4.0
ETA_A = 2.0

NC = 2
NS = 16
NW = NC * NS
NBW = 8

_COS_COEF = [(-1.0) ** k * math.pi ** (2 * k) / math.factorial(2 * k)
             for k in range(8)]

_EC = 256
_NEC = E // _EC
_RSTEPS = 98
_TC_ = 128
_NTC = T // _TC_
_ASTEPS = 245
_FR = 200
_NFC = N // _FR
_NFS = (_NFC + NS - 1) // NS


def _rsqrt_newton(s):
    i = lax.bitcast_convert_type(s, jnp.int32)
    i = jnp.int32(0x5F3759DF) - (i >> 1)
    y = lax.bitcast_convert_type(i, jnp.float32)
    for _ in range(3):
        y = y * (1.5 - 0.5 * s * y * y)
    return y


def _fc_poly(d):
    v = d * d * (1.0 / (RC * RC))
    p = jnp.full((16,), _COS_COEF[7], jnp.float32)
    for c in _COS_COEF[6::-1]:
        p = p * v + c
    return jnp.where(d < RC, 0.5 * (p + 1.0), jnp.zeros((16,), jnp.float32))


def _fill_zero(zbuf):
    lanes = jnp.arange(16, dtype=jnp.int32)
    rowpat = lanes // NBW
    colpat = lanes % NBW
    z16 = jnp.zeros((16,), jnp.float32)
    for i in range(_FR * NBW // 16):
        plsc.store_scatter(zbuf, [rowpat + i * (16 // NBW), colpat], z16)


def _zero_acc(acc, zbuf, s):
    for k in range(_NFS):
        f = s + k * NS

        @pl.when(f < _NFC)
        def _go():
            o = pl.multiple_of(f * _FR, _FR)
            pltpu.sync_copy(zbuf, acc.at[pl.ds(o, _FR), :])


def _flush_acc(acc, fbuf, s, out_ref):
    for k in range(_NFS):
        f = s + k * NS

        @pl.when(f < _NFC)
        def _go():
            o = pl.multiple_of(f * _FR, _FR)
            pltpu.sync_copy(acc.at[pl.ds(o, _FR), :], fbuf)
            pltpu.sync_copy(fbuf, out_ref.at[pl.ds(o, _FR), :])


def _radial_body(rxyz, eidx, z, wr, mur, out, rec,
                 acc, wr_v, mu_v,
                 in0, in1, id0, id1, zj0, zj1, ri0, ri1,
                 stage, rec_st, fbuf, zbuf,
                 semi0, semi1, semz0, semz1, semr0, semr1, semx0, semx1):
    c = lax.axis_index("c")
    s = lax.axis_index("s")
    wid = s * NC + c
    inb = (in0, in1)
    idb = (id0, id1)
    zjb = (zj0, zj1)
    rib = (ri0, ri1)
    semi = (semi0, semi1)
    semz = (semz0, semz1)
    semr = (semr0, semr1)
    semx = (semx0, semx1)

    pltpu.sync_copy(wr, wr_v)
    pltpu.sync_copy(mur, mu_v)
    _fill_zero(zbuf)
    lanes = jnp.arange(16, dtype=jnp.int32)

    def issue_idx(g, b):
        gc = jnp.minimum(g, _NEC - 1)
        pltpu.async_copy(eidx.at[gc], idb[b], semx[b])

    def wait_idx(b):
        pltpu.make_async_copy(eidx.at[0], idb[b], semx[b]).wait()

    def issue_data(g, b, h):
        gc = jnp.minimum(g, _NEC - 1)

        @pl.when(h == 0)
        def _i0():
            pltpu.async_copy(rxyz.at[gc], inb[b], semi[b])
            pltpu.async_copy(z.at[idb[b].at[0, 0]], zjb[b].at[0], semz[b])
            pltpu.async_copy(z.at[idb[b].at[1, 0]], zjb[b].at[1], semz[b])

        @pl.when(h != 0)
        def _i1():
            pltpu.async_copy(rec.at[pl.ds(gc * _EC, _EC), :], rib[b], semr[b])

    def wait_data(b, h):
        @pl.when(h == 0)
        def _w0():
            pltpu.make_async_copy(rxyz.at[0], inb[b], semi[b]).wait()
            pltpu.make_async_copy(z.at[idb[b].at[0, 0]], zjb[b].at[0],
                                  semz[b]).wait()
            pltpu.make_async_copy(z.at[idb[b].at[1, 0]], zjb[b].at[1],
                                  semz[b]).wait()

        @pl.when(h != 0)
        def _w1():
            pltpu.make_async_copy(rec.at[pl.ds(0, _EC), :], rib[b],
                                  semr[b]).wait()

    def basis(d, fc, zj, rows, h, mu16):
        for bb in range(NBW):
            w = plsc.load_gather(wr_v, [zj * NB_R + (h * NBW + bb)])
            t = d - mu16[bb]
            val = jnp.exp(-ETA_R * t * t) * fc * w
            plsc.store_scatter(
                stage, [rows, jnp.full((16,), bb, jnp.int32)], val)

    def compute(g, h, mu16, b):
        ib, db, zb, rb = inb[b], idb[b], zjb[b], rib[b]
        for sub in range(2):
            @pl.when(h == 0)
            def _c0():
                for grp in range(8):
                    o = sub * 128 + grp * 16
                    rows = lanes + o
                    x = ib[0, pl.ds(o, 16)]
                    y = ib[1, pl.ds(o, 16)]
                    q = ib[2, pl.ds(o, 16)]
                    s2 = x * x + y * y + q * q
                    d = s2 * _rsqrt_newton(s2)
                    fc = _fc_poly(d)
                    zj = zb[sub, pl.ds(grp * 16, 16)]
                    for f, val in enumerate(
                            (x, y, q, d, fc,
                             lax.bitcast_convert_type(zj, jnp.float32))):
                        plsc.store_scatter(
                            rec_st, [rows, jnp.full((16,), f, jnp.int32)], val)
                    basis(d, fc, zj, rows, h, mu16)

            @pl.when(h != 0)
            def _c1():
                for grp in range(8):
                    o = sub * 128 + grp * 16
                    rows = lanes + o
                    d = plsc.load_gather(
                        rb, [rows, jnp.full((16,), 3, jnp.int32)])
                    fc = plsc.load_gather(
                        rb, [rows, jnp.full((16,), 4, jnp.int32)])
                    zjf = plsc.load_gather(
                        rb, [rows, jnp.full((16,), 5, jnp.int32)])
                    zj = lax.bitcast_convert_type(zjf, jnp.int32)
                    basis(d, fc, zj, rows, h, mu16)
            pltpu.sync_copy(stage.at[pl.ds(sub * 128, 128), :],
                            acc.at[db.at[sub, 1]], add=True)

        @pl.when(h == 0)
        def _drec():
            pltpu.sync_copy(rec_st, rec.at[pl.ds(g * _EC, _EC), :])

    def half_loop(h, carry):
        _zero_acc(acc, zbuf, s)
        plsc.subcore_barrier()
        mu16 = mu_v[pl.ds(pl.multiple_of(h * NBW, NBW), 16)]
        pltpu.sync_copy(eidx.at[wid], idb[0])
        issue_data(wid, 0, h)
        issue_idx(wid + NW, 1)

        def step(t2, cc):
            for b in range(2):
                t = 2 * t2 + b
                g = wid + NW * t
                wait_idx(1 - b)
                issue_data(wid + NW * (t + 1), 1 - b, h)
                wait_data(b, h)

                @pl.when(g < _NEC)
                def _go():
                    compute(g, h, mu16, b)
                issue_idx(wid + NW * (t + 2), b)
            return cc
        lax.fori_loop(0, _RSTEPS, step, 0)
        wait_data(0, h)
        wait_idx(1)
        plsc.subcore_barrier()
        _flush_acc(acc, fbuf, s, out.at[h, c])
        plsc.subcore_barrier()
        return carry

    lax.fori_loop(0, NB_R // NBW, half_loop, 0)


def _angular_body(rec, aidx, wa, mua, outa,
                  acc, wa_v, mu_v,
                  id0, id1, rj0, rj1, rk0, rk1,
                  stage, fbuf, zbuf,
                  sem0, sem1, semx0, semx1):
    c = lax.axis_index("c")
    s = lax.axis_index("s")
    wid = s * NC + c
    idb = (id0, id1)
    rjb = (rj0, rj1)
    rkb = (rk0, rk1)
    sems = (sem0, sem1)
    semx = (semx0, semx1)

    pltpu.sync_copy(wa, wa_v)
    pltpu.sync_copy(mua, mu_v)
    _fill_zero(zbuf)
    lanes = jnp.arange(16, dtype=jnp.int32)

    def issue_idx(g, b):
        gc = jnp.minimum(g, _NTC - 1)
        pltpu.async_copy(aidx.at[gc], idb[b], semx[b])

    def wait_idx(b):
        pltpu.make_async_copy(aidx.at[0], idb[b], semx[b]).wait()

    def issue_data(g, b):
        pltpu.async_copy(rec.at[idb[b].at[0]], rjb[b], sems[b])
        pltpu.async_copy(rec.at[idb[b].at[1]], rkb[b], sems[b])

    def wait_data(b):
        pltpu.make_async_copy(rec.at[idb[b].at[0]], rjb[b], sems[b]).wait()
        pltpu.make_async_copy(rec.at[idb[b].at[1]], rkb[b], sems[b]).wait()

    def compute(h, mu16, b):
        recj, reck = rjb[b], rkb[b]
        for grp in range(8):
            o = grp * 16
            rows = lanes + o

            def fld(r, f):
                return plsc.load_gather(
                    r, [rows, jnp.full((16,), f, jnp.int32)])
            xj, yj, qj, dj, fcj, zjf = (fld(recj, f) for f in range(6))
            xk, yk, qk, dk, fck, zkf = (fld(reck, f) for f in range(6))
            dot = xj * xk + yj * yk + qj * qk
            cos_t = dot / (dj * dk)
            a = 1.0 + cos_t
            a2 = a * a
            a4 = a2 * a2
            ang = (a4 * a4) * (2.0 ** (1.0 - 8.0))
            davg = 0.5 * (dj + dk)
            scale = ang * fcj * fck
            zj = lax.bitcast_convert_type(zjf, jnp.int32)
            zk = lax.bitcast_convert_type(zkf, jnp.int32)
            for bb in range(NBW):
                wj = plsc.load_gather(wa_v, [zj * NB_A + (h * NBW + bb)])
                wk = plsc.load_gather(wa_v, [zk * NB_A + (h * NBW + bb)])
                t = davg - mu16[bb]
                val = scale * jnp.exp(-ETA_A * t * t) * wj * wk
                plsc.store_scatter(
                    stage, [rows, jnp.full((16,), bb, jnp.int32)], val)
        pltpu.sync_copy(stage, acc.at[idb[b].at[2]], add=True)

    def half_loop(h, carry):
        _zero_acc(acc, zbuf, s)
        plsc.subcore_barrier()
        mu16 = mu_v[pl.ds(pl.multiple_of(h * NBW, NBW), 16)]
        pltpu.sync_copy(aidx.at[wid], idb[0])
        issue_data(wid, 0)
        issue_idx(wid + NW, 1)

        def step(t2, cc):
            for b in range(2):
                t = 2 * t2 + b
                g = wid + NW * t
                wait_idx(1 - b)
                issue_data(wid + NW * (t + 1), 1 - b)
                wait_data(b)

                @pl.when(g < _NTC)
                def _go():
                    compute(h, mu16, b)
                issue_idx(wid + NW * (t + 2), b)
            return cc
        lax.fori_loop(0, _ASTEPS, step, 0)
        wait_data(0)
        wait_idx(1)
        plsc.subcore_barrier()
        _flush_acc(acc, fbuf, s, outa.at[h, c])
        plsc.subcore_barrier()
        return carry

    lax.fori_loop(0, NB_A // NBW, half_loop, 0)


_mesh = plsc.VectorSubcoreMesh(core_axis_name="c", subcore_axis_name="s")

_radial = functools.partial(
    pl.kernel,
    out_type=[
        jax.ShapeDtypeStruct((NB_R // NBW, NC, N, NBW), jnp.float32),
        jax.ShapeDtypeStruct((E, 8), jnp.float32),
    ],
    mesh=_mesh,
    compiler_params=pltpu.CompilerParams(
        needs_layout_passes=False, use_tc_tiling_on_sc=False),
    scratch_types=[
        pltpu.VMEM_SHARED((N, NBW), jnp.float32),
        pltpu.VMEM((N_ELEM * NB_R,), jnp.float32),
        pltpu.VMEM((NB_R + 8,), jnp.float32),
        pltpu.VMEM((3, _EC), jnp.float32),
        pltpu.VMEM((3, _EC), jnp.float32),
        pltpu.VMEM((2, 2, 128), jnp.int32),
        pltpu.VMEM((2, 2, 128), jnp.int32),
        pltpu.VMEM((2, 128), jnp.int32),
        pltpu.VMEM((2, 128), jnp.int32),
        pltpu.VMEM((_EC, 8), jnp.float32),
        pltpu.VMEM((_EC, 8), jnp.float32),
        pltpu.VMEM((_EC, NBW), jnp.float32),
        pltpu.VMEM((_EC, 8), jnp.float32),
        pltpu.VMEM((_FR, NBW), jnp.float32),
        pltpu.VMEM((_FR, NBW), jnp.float32),
        pltpu.SemaphoreType.DMA,
        pltpu.SemaphoreType.DMA,
        pltpu.SemaphoreType.DMA,
        pltpu.SemaphoreType.DMA,
        pltpu.SemaphoreType.DMA,
        pltpu.SemaphoreType.DMA,
        pltpu.SemaphoreType.DMA,
        pltpu.SemaphoreType.DMA,
    ],
)(_radial_body)

_angular = functools.partial(
    pl.kernel,
    out_type=[
        jax.ShapeDtypeStruct((NB_A // NBW, NC, N, NBW), jnp.float32),
    ],
    mesh=_mesh,
    compiler_params=pltpu.CompilerParams(
        needs_layout_passes=False, use_tc_tiling_on_sc=False),
    scratch_types=[
        pltpu.VMEM_SHARED((N, NBW), jnp.float32),
        pltpu.VMEM((N_ELEM * NB_A,), jnp.float32),
        pltpu.VMEM((NB_A + 8,), jnp.float32),
        pltpu.VMEM((3, 128), jnp.int32),
        pltpu.VMEM((3, 128), jnp.int32),
        pltpu.VMEM((_TC_, 8), jnp.float32),
        pltpu.VMEM((_TC_, 8), jnp.float32),
        pltpu.VMEM((_TC_, 8), jnp.float32),
        pltpu.VMEM((_TC_, 8), jnp.float32),
        pltpu.VMEM((_TC_, NBW), jnp.float32),
        pltpu.VMEM((_FR, NBW), jnp.float32),
        pltpu.VMEM((_FR, NBW), jnp.float32),
        pltpu.SemaphoreType.DMA,
        pltpu.SemaphoreType.DMA,
        pltpu.SemaphoreType.DMA,
        pltpu.SemaphoreType.DMA,
    ],
)(_angular_body)


def _combine_body(rad, ang, mean, std, out):
    for p in range(NB_R // NBW):
        out[p, 0] = (rad[p, 0, 0] + rad[p, 1, 0] - mean[p]) / std[p]
    for p in range(NB_A // NBW):
        q = NB_R // NBW + p
        out[q, 0] = (ang[p, 0, 0] + ang[p, 1, 0] - mean[q]) / std[q]


_NP = NB_R // NBW + NB_A // NBW
_N16 = N // 16
_combine = pl.pallas_call(
    _combine_body,
    grid=(50,),
    in_specs=[
        pl.BlockSpec((NB_R // NBW, NC, 1, 125, 128),
                     lambda i: (0, 0, i, 0, 0)),
        pl.BlockSpec((NB_A // NBW, NC, 1, 125, 128),
                     lambda i: (0, 0, i, 0, 0)),
        pl.BlockSpec((_NP, 1, 128), lambda i: (0, 0, 0)),
        pl.BlockSpec((_NP, 1, 128), lambda i: (0, 0, 0)),
    ],
    out_specs=pl.BlockSpec((_NP, 1, 125, 128), lambda i: (0, i, 0, 0)),
    out_shape=jax.ShapeDtypeStruct((_NP, 50, 125, 128), jnp.float32),
)


@jax.jit
def kernel(Z, Rij, idx_i, idx_j, idx_i_triples, idx_j_triples, idx_k_triples,
           radial_mu, elem_w_radial, angular_mu, elem_w_angular,
           symfunc_mean, symfunc_stddev):
    rijt = Rij.astype(jnp.float32)
    rxyz = rijt.reshape(E // _EC, _EC, 3).transpose(0, 2, 1)
    z = Z.astype(jnp.int32)
    eidx = jnp.stack(
        [idx_j.astype(jnp.int32).reshape(E // _EC, 2, 128),
         idx_i.astype(jnp.int32).reshape(E // _EC, 2, 128)], axis=2)
    aidx = jnp.stack(
        [idx_j_triples.astype(jnp.int32).reshape(T // _TC_, 128),
         idx_k_triples.astype(jnp.int32).reshape(T // _TC_, 128),
         idx_i_triples.astype(jnp.int32).reshape(T // _TC_, 128)], axis=1)
    mur = jnp.pad(radial_mu.astype(jnp.float32), (0, 8))
    mua = jnp.pad(angular_mu.astype(jnp.float32), (0, 8))

    rad, rec = _radial(rxyz, eidx, z,
                       elem_w_radial.reshape(-1).astype(jnp.float32), mur)
    (ang,) = _angular(rec, aidx,
                      elem_w_angular.reshape(-1).astype(jnp.float32), mua)
    mean = symfunc_mean.astype(jnp.float32).reshape(6, 8)
    std = symfunc_stddev.astype(jnp.float32).reshape(6, 8)
    meanp = jnp.tile(mean[:, None, :], (1, 16, 1)).reshape(6, 1, 128)
    stdp = jnp.tile(std[:, None, :], (1, 16, 1)).reshape(6, 1, 128)
    xp = _combine(rad.reshape(NB_R // NBW, NC, 50, 125, 128),
                  ang.reshape(NB_A // NBW, NC, 50, 125, 128),
                  meanp, stdp)
    return xp.reshape(6, _N16, 16, 8).transpose(1, 2, 0, 3).reshape(N, 48)


# speedup vs baseline: 6.8995x; 1.0125x over previous
import functools
import math

import jax
import jax.numpy as jnp
from jax import lax
from jax.experimental import pallas as pl
from jax.experimental.pallas import tpu as pltpu
from jax.experimental.pallas import tpu_sc as plsc

N = 100000
E = 1600000
T = 2000000
NB_R = 32
NB_A = 16
N_ELEM = 100
RC = 5.0
ETA_R = 4.0
ETA_A = 2.0

NC = 2
NS = 16
NW = NC * NS
NBW = 8

_COS_COEF = [(-1.0) ** k * math.pi ** (2 * k) / math.factorial(2 * k)
             for k in range(8)]

_EC = 256
_NEC = E // _EC
_RSTEPS = 98
_TC_ = 128
_NTC = T // _TC_
_ASTEPS = 245
_FR = 200
_NFC = N // _FR
_NFS = (_NFC + NS - 1) // NS


def _rsqrt_newton(s):
    i = lax.bitcast_convert_type(s, jnp.int32)
    i = jnp.int32(0x5F3759DF) - (i >> 1)
    y = lax.bitcast_convert_type(i, jnp.float32)
    for _ in range(3):
        y = y * (1.5 - 0.5 * s * y * y)
    return y


def _fc_poly(d):
    v = d * d * (1.0 / (RC * RC))
    p = jnp.full((16,), _COS_COEF[7], jnp.float32)
    for c in _COS_COEF[6::-1]:
        p = p * v + c
    return jnp.where(d < RC, 0.5 * (p + 1.0), jnp.zeros((16,), jnp.float32))


def _fill_zero(zbuf):
    lanes = jnp.arange(16, dtype=jnp.int32)
    rowpat = lanes // NBW
    colpat = lanes % NBW
    z16 = jnp.zeros((16,), jnp.float32)
    for i in range(_FR * NBW // 16):
        plsc.store_scatter(zbuf, [rowpat + i * (16 // NBW), colpat], z16)


def _zero_acc(acc, zbuf, s):
    for k in range(_NFS):
        f = s + k * NS

        @pl.when(f < _NFC)
        def _go():
            o = pl.multiple_of(f * _FR, _FR)
            pltpu.sync_copy(zbuf, acc.at[pl.ds(o, _FR), :])


def _flush_acc(acc, fbuf, s, out_ref):
    for k in range(_NFS):
        f = s + k * NS

        @pl.when(f < _NFC)
        def _go():
            o = pl.multiple_of(f * _FR, _FR)
            pltpu.sync_copy(acc.at[pl.ds(o, _FR), :], fbuf)
            pltpu.sync_copy(fbuf, out_ref.at[pl.ds(o, _FR), :])


def _radial_body(rxyz, eidx, z, wr, mur, out, rec,
                 acc, wr_v, mu_v,
                 in0, in1, id0, id1, zj0, zj1, ri0, ri1,
                 stg0, stg1, six0, six1, rec_st, fbuf, zbuf,
                 semi0, semi1, semz0, semz1, semr0, semr1, semx0, semx1,
                 sc0, sc1):
    c = lax.axis_index("c")
    s = lax.axis_index("s")
    wid = s * NC + c
    inb = (in0, in1)
    idb = (id0, id1)
    zjb = (zj0, zj1)
    rib = (ri0, ri1)
    semi = (semi0, semi1)
    semz = (semz0, semz1)
    semr = (semr0, semr1)
    semx = (semx0, semx1)
    stg_b = (stg0, stg1)
    six_b = (six0, six1)
    sems2 = (sc0, sc1)

    pltpu.sync_copy(wr, wr_v)
    pltpu.sync_copy(mur, mu_v)
    _fill_zero(zbuf)
    lanes = jnp.arange(16, dtype=jnp.int32)

    def issue_idx(g, b):
        gc = jnp.minimum(g, _NEC - 1)
        pltpu.async_copy(eidx.at[gc], idb[b], semx[b])

    def wait_idx(b):
        pltpu.make_async_copy(eidx.at[0], idb[b], semx[b]).wait()

    def issue_data(g, b, h):
        gc = jnp.minimum(g, _NEC - 1)

        @pl.when(h == 0)
        def _i0():
            pltpu.async_copy(rxyz.at[gc], inb[b], semi[b])
            pltpu.async_copy(z.at[idb[b].at[0, 0]], zjb[b].at[0], semz[b])
            pltpu.async_copy(z.at[idb[b].at[1, 0]], zjb[b].at[1], semz[b])

        @pl.when(h != 0)
        def _i1():
            pltpu.async_copy(rec.at[pl.ds(gc * _EC, _EC), :], rib[b], semr[b])

    def wait_data(b, h):
        @pl.when(h == 0)
        def _w0():
            pltpu.make_async_copy(rxyz.at[0], inb[b], semi[b]).wait()
            pltpu.make_async_copy(z.at[idb[b].at[0, 0]], zjb[b].at[0],
                                  semz[b]).wait()
            pltpu.make_async_copy(z.at[idb[b].at[1, 0]], zjb[b].at[1],
                                  semz[b]).wait()

        @pl.when(h != 0)
        def _w1():
            pltpu.make_async_copy(rec.at[pl.ds(0, _EC), :], rib[b],
                                  semr[b]).wait()

    def basis(d, fc, zj, rows, h, mu16, stg):
        for bb in range(NBW):
            w = plsc.load_gather(wr_v, [zj * NB_R + (h * NBW + bb)])
            t = d - mu16[bb]
            val = jnp.exp(-ETA_R * t * t) * fc * w
            plsc.store_scatter(
                stg, [rows, jnp.full((16,), bb, jnp.int32)], val)

    def wait_sc(b):
        for sub in range(2):
            pltpu.make_async_copy(
                stg_b[b].at[pl.ds(sub * 128, 128), :],
                acc.at[six_b[b].at[sub]], sems2[b]).wait()

    def compute(g, t, h, mu16, b):
        ib, db, zb, rb = inb[b], idb[b], zjb[b], rib[b]
        stg, six = stg_b[b], six_b[b]

        @pl.when(t >= 2)
        def _wsc():
            wait_sc(b)
        for sub in range(2):
            @pl.when(h == 0)
            def _c0():
                for grp in range(8):
                    o = sub * 128 + grp * 16
                    rows = lanes + o
                    x = ib[0, pl.ds(o, 16)]
                    y = ib[1, pl.ds(o, 16)]
                    q = ib[2, pl.ds(o, 16)]
                    s2 = x * x + y * y + q * q
                    d = s2 * _rsqrt_newton(s2)
                    fc = _fc_poly(d)
                    zj = zb[sub, pl.ds(grp * 16, 16)]
                    for f, val in enumerate(
                            (x, y, q, d, fc,
                             lax.bitcast_convert_type(zj, jnp.float32))):
                        plsc.store_scatter(
                            rec_st, [rows, jnp.full((16,), f, jnp.int32)], val)
                    basis(d, fc, zj, rows, h, mu16, stg)

            @pl.when(h != 0)
            def _c1():
                for grp in range(8):
                    o = sub * 128 + grp * 16
                    rows = lanes + o
                    d = plsc.load_gather(
                        rb, [rows, jnp.full((16,), 3, jnp.int32)])
                    fc = plsc.load_gather(
                        rb, [rows, jnp.full((16,), 4, jnp.int32)])
                    zjf = plsc.load_gather(
                        rb, [rows, jnp.full((16,), 5, jnp.int32)])
                    zj = lax.bitcast_convert_type(zjf, jnp.int32)
                    basis(d, fc, zj, rows, h, mu16, stg)
            for k in range(8):
                six[sub, pl.ds(k * 16, 16)] = db[sub, 1, pl.ds(k * 16, 16)]
            pltpu.async_copy(stg.at[pl.ds(sub * 128, 128), :],
                             acc.at[six.at[sub]], sems2[b], add=True)

        @pl.when(h == 0)
        def _drec():
            pltpu.sync_copy(rec_st, rec.at[pl.ds(g * _EC, _EC), :])

    def half_loop(h, carry):
        _zero_acc(acc, zbuf, s)
        plsc.subcore_barrier()
        mu16 = mu_v[pl.ds(pl.multiple_of(h * NBW, NBW), 16)]
        pltpu.sync_copy(eidx.at[wid], idb[0])
        issue_data(wid, 0, h)
        issue_idx(wid + NW, 1)

        def step(t2, cc):
            for b in range(2):
                t = 2 * t2 + b
                g = wid + NW * t
                wait_idx(1 - b)
                issue_data(wid + NW * (t + 1), 1 - b, h)
                wait_data(b, h)

                @pl.when(g < _NEC)
                def _go():
                    compute(g, t, h, mu16, b)
                issue_idx(wid + NW * (t + 2), b)
            return cc
        lax.fori_loop(0, _RSTEPS, step, 0)
        wait_data(0, h)
        wait_idx(1)
        cnt = (_NEC - wid + NW - 1) // NW

        @pl.when(cnt >= 2)
        def _d2():
            wait_sc(0)
            wait_sc(1)

        @pl.when(cnt == 1)
        def _d1():
            wait_sc(0)
        plsc.subcore_barrier()
        _flush_acc(acc, fbuf, s, out.at[h, c])
        plsc.subcore_barrier()
        return carry

    lax.fori_loop(0, NB_R // NBW, half_loop, 0)


def _angular_body(rec, aidx, wa, mua, outa,
                  acc, wa_v, mu_v,
                  id0, id1, rj0, rj1, rk0, rk1,
                  stg0, stg1, six0, six1, fbuf, zbuf,
                  sem0, sem1, semx0, semx1, sc0, sc1):
    c = lax.axis_index("c")
    s = lax.axis_index("s")
    wid = s * NC + c
    idb = (id0, id1)
    rjb = (rj0, rj1)
    rkb = (rk0, rk1)
    sems = (sem0, sem1)
    semx = (semx0, semx1)
    stg_b = (stg0, stg1)
    six_b = (six0, six1)
    sems2 = (sc0, sc1)

    pltpu.sync_copy(wa, wa_v)
    pltpu.sync_copy(mua, mu_v)
    _fill_zero(zbuf)
    lanes = jnp.arange(16, dtype=jnp.int32)

    def issue_idx(g, b):
        gc = jnp.minimum(g, _NTC - 1)
        pltpu.async_copy(aidx.at[gc], idb[b], semx[b])

    def wait_idx(b):
        pltpu.make_async_copy(aidx.at[0], idb[b], semx[b]).wait()

    def issue_data(g, b):
        pltpu.async_copy(rec.at[idb[b].at[0]], rjb[b], sems[b])
        pltpu.async_copy(rec.at[idb[b].at[1]], rkb[b], sems[b])

    def wait_data(b):
        pltpu.make_async_copy(rec.at[idb[b].at[0]], rjb[b], sems[b]).wait()
        pltpu.make_async_copy(rec.at[idb[b].at[1]], rkb[b], sems[b]).wait()

    def wait_sc(b):
        pltpu.make_async_copy(stg_b[b], acc.at[six_b[b].at[0]],
                              sems2[b]).wait()

    def compute(t, h, mu16, b):
        recj, reck = rjb[b], rkb[b]
        stage, six = stg_b[b], six_b[b]

        @pl.when(t >= 2)
        def _wsc():
            wait_sc(b)
        for grp in range(8):
            o = grp * 16
            rows = lanes + o

            def fld(r, f):
                return plsc.load_gather(
                    r, [rows, jnp.full((16,), f, jnp.int32)])
            xj, yj, qj, dj, fcj, zjf = (fld(recj, f) for f in range(6))
            xk, yk, qk, dk, fck, zkf = (fld(reck, f) for f in range(6))
            dot = xj * xk + yj * yk + qj * qk
            cos_t = dot / (dj * dk)
            a = 1.0 + cos_t
            a2 = a * a
            a4 = a2 * a2
            ang = (a4 * a4) * (2.0 ** (1.0 - 8.0))
            davg = 0.5 * (dj + dk)
            scale = ang * fcj * fck
            zj = lax.bitcast_convert_type(zjf, jnp.int32)
            zk = lax.bitcast_convert_type(zkf, jnp.int32)
            for bb in range(NBW):
                wj = plsc.load_gather(wa_v, [zj * NB_A + (h * NBW + bb)])
                wk = plsc.load_gather(wa_v, [zk * NB_A + (h * NBW + bb)])
                t = davg - mu16[bb]
                val = scale * jnp.exp(-ETA_A * t * t) * wj * wk
                plsc.store_scatter(
                    stage, [rows, jnp.full((16,), bb, jnp.int32)], val)
        for k in range(8):
            six[0, pl.ds(k * 16, 16)] = idb[b][2, pl.ds(k * 16, 16)]
        pltpu.async_copy(stage, acc.at[six.at[0]], sems2[b], add=True)

    def half_loop(h, carry):
        _zero_acc(acc, zbuf, s)
        plsc.subcore_barrier()
        mu16 = mu_v[pl.ds(pl.multiple_of(h * NBW, NBW), 16)]
        pltpu.sync_copy(aidx.at[wid], idb[0])
        issue_data(wid, 0)
        issue_idx(wid + NW, 1)

        def step(t2, cc):
            for b in range(2):
                t = 2 * t2 + b
                g = wid + NW * t
                wait_idx(1 - b)
                issue_data(wid + NW * (t + 1), 1 - b)
                wait_data(b)

                @pl.when(g < _NTC)
                def _go():
                    compute(t, h, mu16, b)
                issue_idx(wid + NW * (t + 2), b)
            return cc
        lax.fori_loop(0, _ASTEPS, step, 0)
        wait_data(0)
        wait_idx(1)
        cnt = (_NTC - wid + NW - 1) // NW

        @pl.when(cnt >= 2)
        def _d2():
            wait_sc(0)
            wait_sc(1)

        @pl.when(cnt == 1)
        def _d1():
            wait_sc(0)
        plsc.subcore_barrier()
        _flush_acc(acc, fbuf, s, outa.at[h, c])
        plsc.subcore_barrier()
        return carry

    lax.fori_loop(0, NB_A // NBW, half_loop, 0)


_mesh = plsc.VectorSubcoreMesh(core_axis_name="c", subcore_axis_name="s")

_radial = functools.partial(
    pl.kernel,
    out_type=[
        jax.ShapeDtypeStruct((NB_R // NBW, NC, N, NBW), jnp.float32),
        jax.ShapeDtypeStruct((E, 8), jnp.float32),
    ],
    mesh=_mesh,
    compiler_params=pltpu.CompilerParams(
        needs_layout_passes=False, use_tc_tiling_on_sc=False),
    scratch_types=[
        pltpu.VMEM_SHARED((N, NBW), jnp.float32),
        pltpu.VMEM((N_ELEM * NB_R,), jnp.float32),
        pltpu.VMEM((NB_R + 8,), jnp.float32),
        pltpu.VMEM((3, _EC), jnp.float32),
        pltpu.VMEM((3, _EC), jnp.float32),
        pltpu.VMEM((2, 2, 128), jnp.int32),
        pltpu.VMEM((2, 2, 128), jnp.int32),
        pltpu.VMEM((2, 128), jnp.int32),
        pltpu.VMEM((2, 128), jnp.int32),
        pltpu.VMEM((_EC, 8), jnp.float32),
        pltpu.VMEM((_EC, 8), jnp.float32),
        pltpu.VMEM((_EC, NBW), jnp.float32),
        pltpu.VMEM((_EC, NBW), jnp.float32),
        pltpu.VMEM((2, 128), jnp.int32),
        pltpu.VMEM((2, 128), jnp.int32),
        pltpu.VMEM((_EC, 8), jnp.float32),
        pltpu.VMEM((_FR, NBW), jnp.float32),
        pltpu.VMEM((_FR, NBW), jnp.float32),
        pltpu.SemaphoreType.DMA,
        pltpu.SemaphoreType.DMA,
        pltpu.SemaphoreType.DMA,
        pltpu.SemaphoreType.DMA,
        pltpu.SemaphoreType.DMA,
        pltpu.SemaphoreType.DMA,
        pltpu.SemaphoreType.DMA,
        pltpu.SemaphoreType.DMA,
        pltpu.SemaphoreType.DMA,
        pltpu.SemaphoreType.DMA,
    ],
)(_radial_body)

_angular = functools.partial(
    pl.kernel,
    out_type=[
        jax.ShapeDtypeStruct((NB_A // NBW, NC, N, NBW), jnp.float32),
    ],
    mesh=_mesh,
    compiler_params=pltpu.CompilerParams(
        needs_layout_passes=False, use_tc_tiling_on_sc=False),
    scratch_types=[
        pltpu.VMEM_SHARED((N, NBW), jnp.float32),
        pltpu.VMEM((N_ELEM * NB_A,), jnp.float32),
        pltpu.VMEM((NB_A + 8,), jnp.float32),
        pltpu.VMEM((3, 128), jnp.int32),
        pltpu.VMEM((3, 128), jnp.int32),
        pltpu.VMEM((_TC_, 8), jnp.float32),
        pltpu.VMEM((_TC_, 8), jnp.float32),
        pltpu.VMEM((_TC_, 8), jnp.float32),
        pltpu.VMEM((_TC_, 8), jnp.float32),
        pltpu.VMEM((_TC_, NBW), jnp.float32),
        pltpu.VMEM((_TC_, NBW), jnp.float32),
        pltpu.VMEM((1, 128), jnp.int32),
        pltpu.VMEM((1, 128), jnp.int32),
        pltpu.VMEM((_FR, NBW), jnp.float32),
        pltpu.VMEM((_FR, NBW), jnp.float32),
        pltpu.SemaphoreType.DMA,
        pltpu.SemaphoreType.DMA,
        pltpu.SemaphoreType.DMA,
        pltpu.SemaphoreType.DMA,
        pltpu.SemaphoreType.DMA,
        pltpu.SemaphoreType.DMA,
    ],
)(_angular_body)


def _combine_body(rad, ang, mean, std, out):
    for p in range(NB_R // NBW):
        out[p, 0] = (rad[p, 0, 0] + rad[p, 1, 0] - mean[p]) / std[p]
    for p in range(NB_A // NBW):
        q = NB_R // NBW + p
        out[q, 0] = (ang[p, 0, 0] + ang[p, 1, 0] - mean[q]) / std[q]


_NP = NB_R // NBW + NB_A // NBW
_N16 = N // 16
_combine = pl.pallas_call(
    _combine_body,
    grid=(50,),
    in_specs=[
        pl.BlockSpec((NB_R // NBW, NC, 1, 125, 128),
                     lambda i: (0, 0, i, 0, 0)),
        pl.BlockSpec((NB_A // NBW, NC, 1, 125, 128),
                     lambda i: (0, 0, i, 0, 0)),
        pl.BlockSpec((_NP, 1, 128), lambda i: (0, 0, 0)),
        pl.BlockSpec((_NP, 1, 128), lambda i: (0, 0, 0)),
    ],
    out_specs=pl.BlockSpec((_NP, 1, 125, 128), lambda i: (0, i, 0, 0)),
    out_shape=jax.ShapeDtypeStruct((_NP, 50, 125, 128), jnp.float32),
)


@jax.jit
def kernel(Z, Rij, idx_i, idx_j, idx_i_triples, idx_j_triples, idx_k_triples,
           radial_mu, elem_w_radial, angular_mu, elem_w_angular,
           symfunc_mean, symfunc_stddev):
    rijt = Rij.astype(jnp.float32)
    rxyz = rijt.reshape(E // _EC, _EC, 3).transpose(0, 2, 1)
    z = Z.astype(jnp.int32)
    eidx = jnp.stack(
        [idx_j.astype(jnp.int32).reshape(E // _EC, 2, 128),
         idx_i.astype(jnp.int32).reshape(E // _EC, 2, 128)], axis=2)
    aidx = jnp.stack(
        [idx_j_triples.astype(jnp.int32).reshape(T // _TC_, 128),
         idx_k_triples.astype(jnp.int32).reshape(T // _TC_, 128),
         idx_i_triples.astype(jnp.int32).reshape(T // _TC_, 128)], axis=1)
    mur = jnp.pad(radial_mu.astype(jnp.float32), (0, 8))
    mua = jnp.pad(angular_mu.astype(jnp.float32), (0, 8))

    rad, rec = _radial(rxyz, eidx, z,
                       elem_w_radial.reshape(-1).astype(jnp.float32), mur)
    (ang,) = _angular(rec, aidx,
                      elem_w_angular.reshape(-1).astype(jnp.float32), mua)
    mean = symfunc_mean.astype(jnp.float32).reshape(6, 8)
    std = symfunc_stddev.astype(jnp.float32).reshape(6, 8)
    meanp = jnp.tile(mean[:, None, :], (1, 16, 1)).reshape(6, 1, 128)
    stdp = jnp.tile(std[:, None, :], (1, 16, 1)).reshape(6, 1, 128)
    xp = _combine(rad.reshape(NB_R // NBW, NC, 50, 125, 128),
                  ang.reshape(NB_A // NBW, NC, 50, 125, 128),
                  meanp, stdp)
    return xp.reshape(6, _N16, 16, 8).transpose(1, 2, 0, 3).reshape(N, 48)
